# Initial kernel scaffold; baseline (speedup 1.0000x reference)
import jax
import jax.numpy as jnp
from jax.experimental import pallas as pl


def kernel(node_features, edge_index_input, edge_prob_input, x, params):
    raise NotImplementedError("write your pallas kernel here")


# speedup vs baseline: 5.0538x; 5.0538x over previous
import functools

import jax
import jax.numpy as jnp
import numpy as np
from jax import lax
from jax.experimental import pallas as pl
from jax.experimental.pallas import tpu as pltpu
from jax.experimental.pallas import tpu_sc as plsc

N = 10000
E = 320000
D = 256
L = 16
NSUB = 16
NCORE = 2
BN = 400
C = 128
EPAD = 327680
ERWS = EPAD // C
EDROWS = 10496
NP = 10240
F32 = jnp.float32

WCH_PT = ERWS // NCORE // NSUB
DROWS = NP // 8
DR_PT = DROWS // NSUB
WROWS = EPAD // 8

PH = 3456
NPH = 3
NP2 = PH * NPH
DUMP = PH
AROWS = PH + 8
PH_PT = PH // NSUB
NCHUNK = ERWS // NSUB


def _tc_pre_body(h_ref, w_ref, wsrc_ref, wtrg_ref, wskip_ref,
                 proj2_ref, sctab_ref, skip_ref, smax_ref):
    i = pl.program_id(0)
    h = h_ref[...]
    proj = jnp.dot(h, w_ref[...], preferred_element_type=F32)
    ssrc = jnp.dot(h, wsrc_ref[...], preferred_element_type=F32)
    strg = jnp.dot(h, wtrg_ref[...], preferred_element_type=F32)
    skip_ref[...] = jnp.dot(h, wskip_ref[...], preferred_element_type=F32)
    proj2_ref[0] = proj[:, :128]
    proj2_ref[1] = proj[:, 128:]
    sctab_ref[...] = jnp.concatenate(
        [ssrc, strg, jnp.zeros((ssrc.shape[0], 128 - 2 * L), F32)], axis=1)

    @pl.when(i == 0)
    def _():
        smax_ref[...] = jnp.full((8, L), -1e30, F32)

    upd = jnp.concatenate(
        [jnp.max(ssrc, axis=0, keepdims=True),
         jnp.max(strg, axis=0, keepdims=True),
         jnp.full((6, L), -1e30, F32)], axis=0)
    smax_ref[...] = jnp.maximum(smax_ref[...], upd)


def _tc_pre(h, W, Wsrc, Wtrg, Wskip):
    fin = h.shape[1]
    nb = N // BN
    out_shapes = [
        jax.ShapeDtypeStruct((2, N, 128), F32),
        jax.ShapeDtypeStruct((N, 128), F32),
        jax.ShapeDtypeStruct((N, D), F32),
        jax.ShapeDtypeStruct((8, L), F32),
    ]
    grid_spec = pl.GridSpec(
        grid=(nb,),
        in_specs=[
            pl.BlockSpec((BN, fin), lambda i: (i, 0)),
            pl.BlockSpec((fin, D), lambda i: (0, 0)),
            pl.BlockSpec((fin, L), lambda i: (0, 0)),
            pl.BlockSpec((fin, L), lambda i: (0, 0)),
            pl.BlockSpec((fin, D), lambda i: (0, 0)),
        ],
        out_specs=[
            pl.BlockSpec((2, BN, 128), lambda i: (0, i, 0)),
            pl.BlockSpec((BN, 128), lambda i: (i, 0)),
            pl.BlockSpec((BN, D), lambda i: (i, 0)),
            pl.BlockSpec((8, L), lambda i: (0, 0)),
        ],
    )
    return pl.pallas_call(_tc_pre_body, grid_spec=grid_spec,
                          out_shape=out_shapes)(h, W, Wsrc, Wtrg, Wskip)


def _tc_post_body(ln, num2_ref, den0_ref, den1_ref, skip_ref, r_ref,
                  bias_ref, gamma_ref, beta_ref, out_ref):
    num = jnp.concatenate([num2_ref[0], num2_ref[1]], axis=1)
    den = den0_ref[...] + den1_ref[...]
    den_exp = jnp.dot(den, r_ref[...], preferred_element_type=F32)
    o = num / (den_exp + 1e-16) + skip_ref[...] + bias_ref[...]
    o = jnp.where(o > 0, o, jnp.exp(jnp.minimum(o, 0.0)) - 1.0)
    if ln:
        mu = jnp.mean(o, axis=1, keepdims=True)
        var = jnp.mean((o - mu) ** 2, axis=1, keepdims=True)
        o = gamma_ref[...] * (o - mu) / jnp.sqrt(var + 1e-5) + beta_ref[...]
    out_ref[...] = o


def _tc_post(num2, den0, den1, skip, R, bias, gamma, beta, ln):
    nb = N // BN
    grid_spec = pl.GridSpec(
        grid=(nb,),
        in_specs=[
            pl.BlockSpec((2, BN, 128), lambda i: (0, i, 0)),
            pl.BlockSpec((BN, L), lambda i: (i, 0)),
            pl.BlockSpec((BN, L), lambda i: (i, 0)),
            pl.BlockSpec((BN, D), lambda i: (i, 0)),
            pl.BlockSpec((L, D), lambda i: (0, 0)),
            pl.BlockSpec((1, D), lambda i: (0, 0)),
            pl.BlockSpec((1, D), lambda i: (0, 0)),
            pl.BlockSpec((1, D), lambda i: (0, 0)),
        ],
        out_specs=pl.BlockSpec((BN, D), lambda i: (i, 0)),
    )
    return pl.pallas_call(functools.partial(_tc_post_body, ln),
                          grid_spec=grid_spec,
                          out_shape=jax.ShapeDtypeStruct((N, D), F32))(
        num2, den0, den1, skip, R, bias, gamma, beta)


def _sc_w_body(sctab_hbm, edata_hbm, par_hbm, w_out, den_out,
               srcv, trgv, didxv, probv, ssrcb, strgb, wb, denb,
               parb, zb, den_sh, sem, sem2):
    c = lax.axis_index("c")
    s = lax.axis_index("s")

    zrow = jnp.zeros((L,), F32)
    def zfill(i, _):
        for k in range(8):
            zb[i, pl.ds(k * L, L)] = zrow
        return 0
    lax.fori_loop(0, 80, zfill, 0)
    pltpu.sync_copy(zb, den_sh.at[pl.ds(s * DR_PT, DR_PT)])

    pltpu.sync_copy(par_hbm, parb)
    plsc.subcore_barrier()

    cvec = parb[pl.ds(0, L)]
    mvec = parb[pl.ds(L, L)]
    zeros_i = jnp.zeros((L,), jnp.int32)
    one_i = zeros_i + 1

    def chunk(i, _):
        row = (c * NSUB + s) * WCH_PT + i
        pltpu.sync_copy(edata_hbm.at[row], srcv)
        pltpu.sync_copy(edata_hbm.at[ERWS + row], trgv)
        pltpu.sync_copy(edata_hbm.at[2 * ERWS + row], probv)
        d1 = pltpu.async_copy(sctab_hbm.at[srcv], ssrcb, sem)
        d2 = pltpu.async_copy(sctab_hbm.at[trgv], strgb, sem2)
        d1.wait()
        d2.wait()
        for k in range(C // L):
            tvec = trgv[pl.ds(k * L, L)]
            didxv[pl.ds(k * L, L)] = lax.div(tvec, 8)

        def ebody(e, _):
            g = lax.div(e, L)
            j = lax.rem(e, L)
            jsp = zeros_i + j
            pvec = probv[pl.ds(g * L, L)]
            psp = pvec.at[jsp].get(mode="promise_in_bounds").astype(F32)
            sv = (ssrcb[e, pl.ds(0, L)] + strgb[e, pl.ds(L, L)]
                  + cvec * psp)
            sv = jnp.maximum(sv, 0.2 * sv) - mvec
            w = jnp.exp(sv)
            wb[lax.div(e, 8), pl.ds(lax.rem(e, 8) * L, L)] = w
            tvec = trgv[pl.ds(g * L, L)]
            tsp = tvec.at[jsp].get(mode="promise_in_bounds")
            msp = lax.rem(tsp, 8)
            for t in range(8):
                eqt = one_i - jnp.minimum(jnp.abs(msp - t), one_i)
                denb[e, pl.ds(t * L, L)] = w * eqt.astype(F32)
            return 0
        lax.fori_loop(0, C, ebody, 0)

        pltpu.sync_copy(denb, den_sh.at[didxv], add=True)
        pltpu.sync_copy(wb, w_out.at[pl.ds(row * (C // 8), C // 8)])
        return 0

    lax.fori_loop(0, WCH_PT, chunk, 0)
    plsc.subcore_barrier()

    pltpu.sync_copy(den_sh.at[pl.ds(s * DR_PT, DR_PT)],
                    den_out.at[pl.ds(c * DROWS + s * DR_PT, DR_PT)])


def _sc_w(sctab, edata, par):
    mesh = plsc.VectorSubcoreMesh(core_axis_name="c", subcore_axis_name="s")
    f = pl.kernel(
        _sc_w_body,
        out_type=[jax.ShapeDtypeStruct((WROWS, 128), F32),
                  jax.ShapeDtypeStruct((2 * DROWS, 128), F32)],
        mesh=mesh,
        scratch_types=[
            pltpu.VMEM((C,), jnp.int32),
            pltpu.VMEM((C,), jnp.int32),
            pltpu.VMEM((C,), jnp.int32),
            pltpu.VMEM((C,), jnp.int32),
            pltpu.VMEM((C, 128), F32),
            pltpu.VMEM((C, 128), F32),
            pltpu.VMEM((C // 8, 128), F32),
            pltpu.VMEM((C, 128), F32),
            pltpu.VMEM((128,), F32),
            pltpu.VMEM((80, 128), F32),
            pltpu.VMEM_SHARED((DROWS, 128), F32),
            pltpu.SemaphoreType.DMA,
            pltpu.SemaphoreType.DMA,
        ],
    )
    return f(sctab, edata, par)


def _sc_n_body(nh, proj2_hbm, edata_hbm, w_hbm, num_out,
               srcv, trgv, rowv, projb, wbuf, numb, zb, acc_sh, sem):
    c = lax.axis_index("c")
    s = lax.axis_index("s")

    zrow = jnp.zeros((L,), F32)
    def zfill(i, _):
        for k in range(8):
            zb[i, pl.ds(k * L, L)] = zrow
        return 0
    lax.fori_loop(0, 80, zfill, 0)

    off = c * N
    lanes = lax.iota(jnp.int32, L)
    zeros_i = jnp.zeros((L,), jnp.int32)
    one_i = zeros_i + 1
    if nh == 1:
        hidx = [zeros_i] * 8
    else:
        hidx = [zeros_i + (c * 8 + k) for k in range(8)]

    for p in range(NPH):
        for (zo, zn) in ((0, 80), (80, 80), (160, 56)):
            pltpu.sync_copy(zb.at[pl.ds(0, zn)],
                            acc_sh.at[pl.ds(s * PH_PT + zo, zn)])
        plsc.subcore_barrier()

        def chunk(i, _):
            row = s * NCHUNK + i
            pltpu.sync_copy(edata_hbm.at[row], srcv)
            pltpu.sync_copy(edata_hbm.at[ERWS + row], trgv)
            pltpu.sync_copy(w_hbm.at[pl.ds(row * (C // 8), C // 8)], wbuf)
            for k in range(C // L):
                tvec = trgv[pl.ds(k * L, L)]
                q = tvec - p * PH
                a = jnp.minimum(jnp.maximum(q, 0), PH - 1)
                eq = one_i - jnp.minimum(jnp.abs(q - a), one_i)
                rowv[pl.ds(k * L, L)] = DUMP + (a - DUMP) * eq
                srcv[pl.ds(k * L, L)] = srcv[pl.ds(k * L, L)] + off
            pltpu.async_copy(proj2_hbm.at[srcv], projb, sem).wait()

            def ebody(e, _):
                w = wbuf[lax.div(e, 8), pl.ds(lax.rem(e, 8) * L, L)]
                for k in range(8):
                    pv = projb[e, pl.ds(k * L, L)]
                    wk = w.at[hidx[k]].get(mode="promise_in_bounds")
                    numb[e, pl.ds(k * L, L)] = pv * wk
                return 0
            lax.fori_loop(0, C, ebody, 0)

            pltpu.sync_copy(numb, acc_sh.at[rowv], add=True)
            return 0

        lax.fori_loop(0, NCHUNK, chunk, 0)
        plsc.subcore_barrier()

        pltpu.sync_copy(acc_sh.at[pl.ds(s * PH_PT, PH_PT)],
                        num_out.at[pl.ds(c * NP2 + p * PH + s * PH_PT,
                                         PH_PT)])
        plsc.subcore_barrier()


def _sc_n(proj2, edata, warr, nh):
    mesh = plsc.VectorSubcoreMesh(core_axis_name="c", subcore_axis_name="s")
    f = pl.kernel(
        functools.partial(_sc_n_body, nh),
        out_type=jax.ShapeDtypeStruct((2 * NP2, 128), F32),
        mesh=mesh,
        scratch_types=[
            pltpu.VMEM((C,), jnp.int32),
            pltpu.VMEM((C,), jnp.int32),
            pltpu.VMEM((C,), jnp.int32),
            pltpu.VMEM((C, 128), F32),
            pltpu.VMEM((C // 8, 128), F32),
            pltpu.VMEM((C, 128), F32),
            pltpu.VMEM((80, 128), F32),
            pltpu.VMEM_SHARED((AROWS, 128), F32),
            pltpu.SemaphoreType.DMA,
        ],
    )
    return f(proj2, edata, warr)


def _sc_gather_body(h_hbm, idx_hbm, out_hbm, idxv, rowsb, sem):
    c = lax.axis_index("c")
    s = lax.axis_index("s")
    wid = s * NCORE + c
    for j in range(2):
        base = wid * 2 + j
        pltpu.sync_copy(idx_hbm.at[base], idxv)
        pltpu.async_copy(h_hbm.at[idxv], rowsb, sem).wait()
        pltpu.sync_copy(rowsb, out_hbm.at[pl.ds(base * 128, 128)])


def _sc_gather(h, idx):
    nidx = idx.shape[0]
    idx = idx.reshape(nidx // 128, 128)
    mesh = plsc.VectorSubcoreMesh(core_axis_name="c", subcore_axis_name="s")
    f = pl.kernel(
        _sc_gather_body,
        out_type=jax.ShapeDtypeStruct((nidx, D), F32),
        mesh=mesh,
        scratch_types=[
            pltpu.VMEM((128,), jnp.int32),
            pltpu.VMEM((128, D), F32),
            pltpu.SemaphoreType.DMA,
        ],
    )
    return f(h, idx)


def _layer(h, edata, p, concat, nh):
    fout = D // nh
    W = p['W']
    ssrc_flat = p['scoring_src'].reshape(-1)
    strg_flat = p['scoring_trg'].reshape(-1)
    S = np.zeros((D, L), np.float32)
    for j in range(D):
        S[j, j // fout] = 1.0
    S = jnp.asarray(S)
    Wsrc = (W * ssrc_flat[None, :]) @ S
    Wtrg = (W * strg_flat[None, :]) @ S
    Wskip = p['W_skip'] if 'W_skip' in p else jnp.eye(h.shape[1], D, dtype=F32)

    proj2, sctab, skip, smax = _tc_pre(h, W, Wsrc, Wtrg, Wskip)

    cvec = (p['W_prob'].reshape(nh, fout) * p['scoring_prob'][0]).sum(-1)
    c16 = jnp.zeros((L,), F32).at[:nh].set(cvec)
    mh = smax[0] + smax[1] + jnp.maximum(c16, 0.0)
    mh = jnp.maximum(mh, 0.2 * mh)
    M = jnp.max(mh[:nh])
    c16 = c16 * (1.0 / 16777216.0)
    par = jnp.concatenate([c16, jnp.full((L,), M, F32),
                           jnp.zeros((128 - 2 * L,), F32)])

    warr, den2 = _sc_w(sctab, edata, par)
    num2 = _sc_n(proj2.reshape(2 * N, 128), edata, warr, nh)

    den2 = den2.reshape(2, NP, L)

    R = np.zeros((L, D), np.float32)
    for j in range(D):
        R[j // fout, j] = 1.0
    R = jnp.asarray(R)
    ln = 'ln_gamma' in p
    gamma = p['ln_gamma'].reshape(1, D) if ln else jnp.zeros((1, D), F32)
    beta = p['ln_beta'].reshape(1, D) if ln else jnp.zeros((1, D), F32)
    return _tc_post(num2.reshape(2, NP2, 128), den2[0], den2[1], skip, R,
                    p['bias'].reshape(1, D), gamma, beta, ln)


def kernel(node_features, edge_index_input, edge_prob_input, x, params):
    npad = EPAD - E
    src = jnp.concatenate([edge_index_input[0].astype(jnp.int32),
                           jnp.zeros((npad,), jnp.int32)])
    trg = jnp.concatenate([edge_index_input[1].astype(jnp.int32),
                           jnp.full((npad,), NP - 1, jnp.int32)])
    prob = jnp.concatenate([edge_prob_input[:, 0],
                            jnp.zeros((npad,), F32)])
    pfix = (prob * 16777216.0).astype(jnp.int32)
    edata = jnp.concatenate([
        src.reshape(ERWS, C), trg.reshape(ERWS, C), pfix.reshape(ERWS, C),
        jnp.zeros((EDROWS - 3 * ERWS, C), jnp.int32)])

    h = _layer(node_features, edata, params['enc'], True, 16)
    h = _layer(h, edata, params['gm0'], True, 16)
    h = _layer(h, edata, params['gm1'], False, 1)

    B, S_ = x.shape
    idx = x.reshape(-1).astype(jnp.int32)
    out = _sc_gather(h, idx)
    return out.reshape(B, S_, D)


# speedup vs baseline: 6.9613x; 1.3774x over previous
import functools

import jax
import jax.numpy as jnp
import numpy as np
from jax import lax
from jax.experimental import pallas as pl
from jax.experimental.pallas import tpu as pltpu
from jax.experimental.pallas import tpu_sc as plsc

N = 10000
E = 320000
D = 256
L = 16
NSUB = 16
NCORE = 2
BN = 400
C = 128
EPAD = 327680
ERWS = EPAD // C
EDROWS = 10496
NP = 10240
F32 = jnp.float32

WCH_PT = ERWS // NCORE // NSUB
DROWS = NP // 8
DR_PT = DROWS // NSUB
WROWS = EPAD // 8

PH = 5120
NPH = 2
NP2 = PH * NPH
DUMP = PH
AROWS = PH + 8
PH_PT = PH // NSUB
NCHUNK = ERWS // NSUB


def _tc_pre_body(h_ref, w_ref, wsrc_ref, wtrg_ref, wskip_ref,
                 proj2_ref, sctab_ref, skip_ref, smax_ref):
    i = pl.program_id(0)
    h = h_ref[...]
    proj = jnp.dot(h, w_ref[...], preferred_element_type=F32)
    ssrc = jnp.dot(h, wsrc_ref[...], preferred_element_type=F32)
    strg = jnp.dot(h, wtrg_ref[...], preferred_element_type=F32)
    skip_ref[...] = jnp.dot(h, wskip_ref[...], preferred_element_type=F32)
    proj2_ref[0] = proj[:, :128]
    proj2_ref[1] = proj[:, 128:]
    sctab_ref[...] = jnp.concatenate(
        [ssrc, strg, jnp.zeros((ssrc.shape[0], 128 - 2 * L), F32)], axis=1)

    @pl.when(i == 0)
    def _():
        smax_ref[...] = jnp.full((8, L), -1e30, F32)

    upd = jnp.concatenate(
        [jnp.max(ssrc, axis=0, keepdims=True),
         jnp.max(strg, axis=0, keepdims=True),
         jnp.full((6, L), -1e30, F32)], axis=0)
    smax_ref[...] = jnp.maximum(smax_ref[...], upd)


def _tc_pre(h, W, Wsrc, Wtrg, Wskip):
    fin = h.shape[1]
    nb = N // BN
    out_shapes = [
        jax.ShapeDtypeStruct((2, N, 128), F32),
        jax.ShapeDtypeStruct((N, 128), F32),
        jax.ShapeDtypeStruct((N, D), F32),
        jax.ShapeDtypeStruct((8, L), F32),
    ]
    grid_spec = pl.GridSpec(
        grid=(nb,),
        in_specs=[
            pl.BlockSpec((BN, fin), lambda i: (i, 0)),
            pl.BlockSpec((fin, D), lambda i: (0, 0)),
            pl.BlockSpec((fin, L), lambda i: (0, 0)),
            pl.BlockSpec((fin, L), lambda i: (0, 0)),
            pl.BlockSpec((fin, D), lambda i: (0, 0)),
        ],
        out_specs=[
            pl.BlockSpec((2, BN, 128), lambda i: (0, i, 0)),
            pl.BlockSpec((BN, 128), lambda i: (i, 0)),
            pl.BlockSpec((BN, D), lambda i: (i, 0)),
            pl.BlockSpec((8, L), lambda i: (0, 0)),
        ],
    )
    return pl.pallas_call(_tc_pre_body, grid_spec=grid_spec,
                          out_shape=out_shapes)(h, W, Wsrc, Wtrg, Wskip)


def _tc_post_body(ln, num2_ref, den0_ref, den1_ref, skip_ref, r_ref,
                  bias_ref, gamma_ref, beta_ref, out_ref):
    num = jnp.concatenate([num2_ref[0], num2_ref[1]], axis=1)
    den = den0_ref[...] + den1_ref[...]
    den_exp = jnp.dot(den, r_ref[...], preferred_element_type=F32)
    o = num / (den_exp + 1e-16) + skip_ref[...] + bias_ref[...]
    o = jnp.where(o > 0, o, jnp.exp(jnp.minimum(o, 0.0)) - 1.0)
    if ln:
        mu = jnp.mean(o, axis=1, keepdims=True)
        var = jnp.mean((o - mu) ** 2, axis=1, keepdims=True)
        o = gamma_ref[...] * (o - mu) / jnp.sqrt(var + 1e-5) + beta_ref[...]
    out_ref[...] = o


def _tc_post(num2, den0, den1, skip, R, bias, gamma, beta, ln):
    nb = N // BN
    grid_spec = pl.GridSpec(
        grid=(nb,),
        in_specs=[
            pl.BlockSpec((2, BN, 128), lambda i: (0, i, 0)),
            pl.BlockSpec((BN, L), lambda i: (i, 0)),
            pl.BlockSpec((BN, L), lambda i: (i, 0)),
            pl.BlockSpec((BN, D), lambda i: (i, 0)),
            pl.BlockSpec((L, D), lambda i: (0, 0)),
            pl.BlockSpec((1, D), lambda i: (0, 0)),
            pl.BlockSpec((1, D), lambda i: (0, 0)),
            pl.BlockSpec((1, D), lambda i: (0, 0)),
        ],
        out_specs=pl.BlockSpec((BN, D), lambda i: (i, 0)),
    )
    return pl.pallas_call(functools.partial(_tc_post_body, ln),
                          grid_spec=grid_spec,
                          out_shape=jax.ShapeDtypeStruct((N, D), F32))(
        num2, den0, den1, skip, R, bias, gamma, beta)


def _sc_w_body(sctab_hbm, edata_hbm, par_hbm, w_out, den_out,
               srcv, trgv, didxv, probv, ssrcb, strgb, wb, denb,
               parb, zb, den_sh, sem, sem2):
    c = lax.axis_index("c")
    s = lax.axis_index("s")

    zrow = jnp.zeros((L,), F32)
    def zfill(i, _):
        for k in range(8):
            zb[i, pl.ds(k * L, L)] = zrow
        return 0
    lax.fori_loop(0, 80, zfill, 0)
    pltpu.sync_copy(zb, den_sh.at[pl.ds(s * DR_PT, DR_PT)])

    pltpu.sync_copy(par_hbm, parb)
    plsc.subcore_barrier()

    cvec = parb[pl.ds(0, L)]
    mvec = parb[pl.ds(L, L)]
    zeros_i = jnp.zeros((L,), jnp.int32)
    one_i = zeros_i + 1

    def chunk(i, _):
        row = (c * NSUB + s) * WCH_PT + i
        pltpu.sync_copy(edata_hbm.at[row], srcv)
        pltpu.sync_copy(edata_hbm.at[ERWS + row], trgv)
        pltpu.sync_copy(edata_hbm.at[2 * ERWS + row], probv)
        d1 = pltpu.async_copy(sctab_hbm.at[srcv], ssrcb, sem)
        d2 = pltpu.async_copy(sctab_hbm.at[trgv], strgb, sem2)
        d1.wait()
        d2.wait()
        for k in range(C // L):
            tvec = trgv[pl.ds(k * L, L)]
            didxv[pl.ds(k * L, L)] = lax.div(tvec, 8)

        def ebody(e, _):
            g = lax.div(e, L)
            j = lax.rem(e, L)
            jsp = zeros_i + j
            pvec = probv[pl.ds(g * L, L)]
            psp = pvec.at[jsp].get(mode="promise_in_bounds").astype(F32)
            sv = (ssrcb[e, pl.ds(0, L)] + strgb[e, pl.ds(L, L)]
                  + cvec * psp)
            sv = jnp.maximum(sv, 0.2 * sv) - mvec
            w = jnp.exp(sv)
            wb[lax.div(e, 8), pl.ds(lax.rem(e, 8) * L, L)] = w
            tvec = trgv[pl.ds(g * L, L)]
            tsp = tvec.at[jsp].get(mode="promise_in_bounds")
            msp = lax.rem(tsp, 8)
            for t in range(8):
                eqt = one_i - jnp.minimum(jnp.abs(msp - t), one_i)
                denb[e, pl.ds(t * L, L)] = w * eqt.astype(F32)
            return 0
        lax.fori_loop(0, C, ebody, 0)

        pltpu.sync_copy(denb, den_sh.at[didxv], add=True)
        pltpu.sync_copy(wb, w_out.at[pl.ds(row * (C // 8), C // 8)])
        return 0

    lax.fori_loop(0, WCH_PT, chunk, 0)
    plsc.subcore_barrier()

    pltpu.sync_copy(den_sh.at[pl.ds(s * DR_PT, DR_PT)],
                    den_out.at[pl.ds(c * DROWS + s * DR_PT, DR_PT)])


_SC_W_CACHE = {}


def _sc_w(sctab, edata, par):
    if "w" in _SC_W_CACHE:
        return _SC_W_CACHE["w"](sctab, edata, par)
    mesh = plsc.VectorSubcoreMesh(core_axis_name="c", subcore_axis_name="s")
    f = pl.kernel(
        _sc_w_body,
        out_type=[jax.ShapeDtypeStruct((WROWS, 128), F32),
                  jax.ShapeDtypeStruct((2 * DROWS, 128), F32)],
        mesh=mesh,
        scratch_types=[
            pltpu.VMEM((C,), jnp.int32),
            pltpu.VMEM((C,), jnp.int32),
            pltpu.VMEM((C,), jnp.int32),
            pltpu.VMEM((C,), jnp.int32),
            pltpu.VMEM((C, 128), F32),
            pltpu.VMEM((C, 128), F32),
            pltpu.VMEM((C // 8, 128), F32),
            pltpu.VMEM((C, 128), F32),
            pltpu.VMEM((128,), F32),
            pltpu.VMEM((80, 128), F32),
            pltpu.VMEM_SHARED((DROWS, 128), F32),
            pltpu.SemaphoreType.DMA,
            pltpu.SemaphoreType.DMA,
        ],
    )
    _SC_W_CACHE["w"] = f
    return f(sctab, edata, par)


def _sc_n_body(nh, proj2_hbm, edata_hbm, w_hbm, num_out,
               srcv, trgv, rowv, projb, wbuf, numb, zb, acc_sh, sem):
    c = lax.axis_index("c")
    s = lax.axis_index("s")

    zrow = jnp.zeros((L,), F32)
    def zfill(i, _):
        for k in range(8):
            zb[i, pl.ds(k * L, L)] = zrow
        return 0
    lax.fori_loop(0, 80, zfill, 0)

    off = c * N
    lanes = lax.iota(jnp.int32, L)
    zeros_i = jnp.zeros((L,), jnp.int32)
    one_i = zeros_i + 1
    if nh == 1:
        hidx = [zeros_i] * 8
    else:
        hidx = [zeros_i + (c * 8 + k) for k in range(8)]

    for p in range(NPH):
        for zo in range(0, PH_PT, 80):
            pltpu.sync_copy(zb, acc_sh.at[pl.ds(s * PH_PT + zo, 80)])
        plsc.subcore_barrier()

        def chunk(i, _):
            row = s * NCHUNK + i
            pltpu.sync_copy(edata_hbm.at[row], srcv)
            pltpu.sync_copy(edata_hbm.at[ERWS + row], trgv)
            pltpu.sync_copy(w_hbm.at[pl.ds(row * (C // 8), C // 8)], wbuf)
            for k in range(C // L):
                tvec = trgv[pl.ds(k * L, L)]
                q = tvec - p * PH
                a = jnp.minimum(jnp.maximum(q, 0), PH - 1)
                eq = one_i - jnp.minimum(jnp.abs(q - a), one_i)
                rowv[pl.ds(k * L, L)] = DUMP + (a - DUMP) * eq
                srcv[pl.ds(k * L, L)] = srcv[pl.ds(k * L, L)] + off
            pltpu.async_copy(proj2_hbm.at[srcv], projb, sem).wait()

            def ebody(e, _):
                w = wbuf[lax.div(e, 8), pl.ds(lax.rem(e, 8) * L, L)]
                for k in range(8):
                    pv = projb[e, pl.ds(k * L, L)]
                    wk = w.at[hidx[k]].get(mode="promise_in_bounds")
                    numb[e, pl.ds(k * L, L)] = pv * wk
                return 0
            lax.fori_loop(0, C, ebody, 0)

            pltpu.sync_copy(numb, acc_sh.at[rowv], add=True)
            return 0

        lax.fori_loop(0, NCHUNK, chunk, 0)
        plsc.subcore_barrier()

        pltpu.sync_copy(acc_sh.at[pl.ds(s * PH_PT, PH_PT)],
                        num_out.at[pl.ds(c * NP2 + p * PH + s * PH_PT,
                                         PH_PT)])
        plsc.subcore_barrier()


def _sc_n(proj2, edata, warr, nh):
    if nh in _SC_W_CACHE:
        return _SC_W_CACHE[nh](proj2, edata, warr)
    mesh = plsc.VectorSubcoreMesh(core_axis_name="c", subcore_axis_name="s")
    f = pl.kernel(
        functools.partial(_sc_n_body, nh),
        out_type=jax.ShapeDtypeStruct((2 * NP2, 128), F32),
        mesh=mesh,
        scratch_types=[
            pltpu.VMEM((C,), jnp.int32),
            pltpu.VMEM((C,), jnp.int32),
            pltpu.VMEM((C,), jnp.int32),
            pltpu.VMEM((C, 128), F32),
            pltpu.VMEM((C // 8, 128), F32),
            pltpu.VMEM((C, 128), F32),
            pltpu.VMEM((80, 128), F32),
            pltpu.VMEM_SHARED((AROWS, 128), F32),
            pltpu.SemaphoreType.DMA,
        ],
    )
    _SC_W_CACHE[nh] = f
    return f(proj2, edata, warr)


def _sc_gather_body(h_hbm, idx_hbm, out_hbm, idxv, rowsb, sem):
    c = lax.axis_index("c")
    s = lax.axis_index("s")
    wid = s * NCORE + c
    for j in range(2):
        base = wid * 2 + j
        pltpu.sync_copy(idx_hbm.at[base], idxv)
        pltpu.async_copy(h_hbm.at[idxv], rowsb, sem).wait()
        pltpu.sync_copy(rowsb, out_hbm.at[pl.ds(base * 128, 128)])


def _sc_gather(h, idx):
    nidx = idx.shape[0]
    idx = idx.reshape(nidx // 128, 128)
    mesh = plsc.VectorSubcoreMesh(core_axis_name="c", subcore_axis_name="s")
    f = pl.kernel(
        _sc_gather_body,
        out_type=jax.ShapeDtypeStruct((nidx, D), F32),
        mesh=mesh,
        scratch_types=[
            pltpu.VMEM((128,), jnp.int32),
            pltpu.VMEM((128, D), F32),
            pltpu.SemaphoreType.DMA,
        ],
    )
    return f(h, idx)


def _layer(h, edata, p, concat, nh):
    fout = D // nh
    W = p['W']
    ssrc_flat = p['scoring_src'].reshape(-1)
    strg_flat = p['scoring_trg'].reshape(-1)
    S = np.zeros((D, L), np.float32)
    for j in range(D):
        S[j, j // fout] = 1.0
    S = jnp.asarray(S)
    Wsrc = (W * ssrc_flat[None, :]) @ S
    Wtrg = (W * strg_flat[None, :]) @ S
    Wskip = p['W_skip'] if 'W_skip' in p else jnp.eye(h.shape[1], D, dtype=F32)

    proj2, sctab, skip, smax = _tc_pre(h, W, Wsrc, Wtrg, Wskip)

    cvec = (p['W_prob'].reshape(nh, fout) * p['scoring_prob'][0]).sum(-1)
    c16 = jnp.zeros((L,), F32).at[:nh].set(cvec)
    mh = smax[0] + smax[1] + jnp.maximum(c16, 0.0)
    mh = jnp.maximum(mh, 0.2 * mh)
    M = jnp.max(mh[:nh])
    c16 = c16 * (1.0 / 16777216.0)
    par = jnp.concatenate([c16, jnp.full((L,), M, F32),
                           jnp.zeros((128 - 2 * L,), F32)])

    warr, den2 = _sc_w(sctab, edata, par)
    num2 = _sc_n(proj2.reshape(2 * N, 128), edata, warr, nh)

    den2 = den2.reshape(2, NP, L)

    R = np.zeros((L, D), np.float32)
    for j in range(D):
        R[j // fout, j] = 1.0
    R = jnp.asarray(R)
    ln = 'ln_gamma' in p
    gamma = p['ln_gamma'].reshape(1, D) if ln else jnp.zeros((1, D), F32)
    beta = p['ln_beta'].reshape(1, D) if ln else jnp.zeros((1, D), F32)
    return _tc_post(num2.reshape(2, NP2, 128), den2[0], den2[1], skip, R,
                    p['bias'].reshape(1, D), gamma, beta, ln)


def kernel(node_features, edge_index_input, edge_prob_input, x, params):
    npad = EPAD - E
    src = jnp.concatenate([edge_index_input[0].astype(jnp.int32),
                           jnp.zeros((npad,), jnp.int32)])
    trg = jnp.concatenate([edge_index_input[1].astype(jnp.int32),
                           jnp.full((npad,), NP - 1, jnp.int32)])
    prob = jnp.concatenate([edge_prob_input[:, 0],
                            jnp.zeros((npad,), F32)])
    pfix = (prob * 16777216.0).astype(jnp.int32)
    edata = jnp.concatenate([
        src.reshape(ERWS, C), trg.reshape(ERWS, C), pfix.reshape(ERWS, C),
        jnp.zeros((EDROWS - 3 * ERWS, C), jnp.int32)])

    h = _layer(node_features, edata, params['enc'], True, 16)
    h = _layer(h, edata, params['gm0'], True, 16)
    h = _layer(h, edata, params['gm1'], False, 1)

    B, S_ = x.shape
    idx = x.reshape(-1).astype(jnp.int32)
    out = _sc_gather(h, idx)
    return out.reshape(B, S_, D)


# speedup vs baseline: 11.3414x; 1.6292x over previous
import functools

import jax
import jax.numpy as jnp
import numpy as np
from jax import lax
from jax.experimental import pallas as pl
from jax.experimental.pallas import tpu as pltpu
from jax.experimental.pallas import tpu_sc as plsc

N = 10000
E = 320000
D = 256
L = 16
NSUB = 16
NCORE = 2
BN = 400
C = 128
EPAD = 327680
ERWS = EPAD // C
EDROWS = 10496
NP = 10240
F32 = jnp.float32

WCH_PT = ERWS // NCORE // NSUB
DROWS = NP // 8
DR_PT = DROWS // NSUB
WROWS = EPAD // 8

PH = 10240
NPH = 1
NP2 = PH * NPH
DUMP = PH
AROWS = PH + 8
PH_PT = PH // NSUB
NCHUNK = ERWS // NSUB


def _tc_pre_body(h_ref, w_ref, wsrc_ref, wtrg_ref, wskip_ref,
                 proj2_ref, sctab_ref, skip_ref, smax_ref):
    i = pl.program_id(0)
    h = h_ref[...]
    proj = jnp.dot(h, w_ref[...], preferred_element_type=F32)
    ssrc = jnp.dot(h, wsrc_ref[...], preferred_element_type=F32)
    strg = jnp.dot(h, wtrg_ref[...], preferred_element_type=F32)
    skip_ref[...] = jnp.dot(h, wskip_ref[...], preferred_element_type=F32)
    proj2_ref[0] = proj[:, :128]
    proj2_ref[1] = proj[:, 128:]
    sctab_ref[...] = jnp.concatenate(
        [ssrc, strg, jnp.zeros((ssrc.shape[0], 128 - 2 * L), F32)], axis=1)

    @pl.when(i == 0)
    def _():
        smax_ref[...] = jnp.full((8, L), -1e30, F32)

    upd = jnp.concatenate(
        [jnp.max(ssrc, axis=0, keepdims=True),
         jnp.max(strg, axis=0, keepdims=True),
         jnp.full((6, L), -1e30, F32)], axis=0)
    smax_ref[...] = jnp.maximum(smax_ref[...], upd)


def _tc_pre(h, W, Wsrc, Wtrg, Wskip):
    fin = h.shape[1]
    nb = N // BN
    out_shapes = [
        jax.ShapeDtypeStruct((2, N, 128), F32),
        jax.ShapeDtypeStruct((N, 128), F32),
        jax.ShapeDtypeStruct((N, D), F32),
        jax.ShapeDtypeStruct((8, L), F32),
    ]
    grid_spec = pl.GridSpec(
        grid=(nb,),
        in_specs=[
            pl.BlockSpec((BN, fin), lambda i: (i, 0)),
            pl.BlockSpec((fin, D), lambda i: (0, 0)),
            pl.BlockSpec((fin, L), lambda i: (0, 0)),
            pl.BlockSpec((fin, L), lambda i: (0, 0)),
            pl.BlockSpec((fin, D), lambda i: (0, 0)),
        ],
        out_specs=[
            pl.BlockSpec((2, BN, 128), lambda i: (0, i, 0)),
            pl.BlockSpec((BN, 128), lambda i: (i, 0)),
            pl.BlockSpec((BN, D), lambda i: (i, 0)),
            pl.BlockSpec((8, L), lambda i: (0, 0)),
        ],
    )
    return pl.pallas_call(_tc_pre_body, grid_spec=grid_spec,
                          out_shape=out_shapes)(h, W, Wsrc, Wtrg, Wskip)


def _tc_post_body(ln, num2_ref, den0_ref, den1_ref, skip_ref, r_ref,
                  bias_ref, gamma_ref, beta_ref, out_ref):
    num = jnp.concatenate([num2_ref[0], num2_ref[1]], axis=1)
    den = den0_ref[...] + den1_ref[...]
    den_exp = jnp.dot(den, r_ref[...], preferred_element_type=F32)
    o = num / (den_exp + 1e-16) + skip_ref[...] + bias_ref[...]
    o = jnp.where(o > 0, o, jnp.exp(jnp.minimum(o, 0.0)) - 1.0)
    if ln:
        mu = jnp.mean(o, axis=1, keepdims=True)
        var = jnp.mean((o - mu) ** 2, axis=1, keepdims=True)
        o = gamma_ref[...] * (o - mu) / jnp.sqrt(var + 1e-5) + beta_ref[...]
    out_ref[...] = o


def _tc_post(num2, den0, den1, skip, R, bias, gamma, beta, ln):
    nb = N // BN
    grid_spec = pl.GridSpec(
        grid=(nb,),
        in_specs=[
            pl.BlockSpec((2, BN, 128), lambda i: (0, i, 0)),
            pl.BlockSpec((BN, L), lambda i: (i, 0)),
            pl.BlockSpec((BN, L), lambda i: (i, 0)),
            pl.BlockSpec((BN, D), lambda i: (i, 0)),
            pl.BlockSpec((L, D), lambda i: (0, 0)),
            pl.BlockSpec((1, D), lambda i: (0, 0)),
            pl.BlockSpec((1, D), lambda i: (0, 0)),
            pl.BlockSpec((1, D), lambda i: (0, 0)),
        ],
        out_specs=pl.BlockSpec((BN, D), lambda i: (i, 0)),
    )
    return pl.pallas_call(functools.partial(_tc_post_body, ln),
                          grid_spec=grid_spec,
                          out_shape=jax.ShapeDtypeStruct((N, D), F32))(
        num2, den0, den1, skip, R, bias, gamma, beta)


def _sc_w_body(sctab_hbm, edata_hbm, par_hbm, w_out, den_out,
               srcv, trgv, didxv, probv, ssrcb, strgb, wb, denb,
               parb, zb, den_sh, sem, sem2):
    c = lax.axis_index("c")
    s = lax.axis_index("s")

    zrow = jnp.zeros((L,), F32)
    def zfill(i, _):
        for k in range(8):
            zb[i, pl.ds(k * L, L)] = zrow
        return 0
    lax.fori_loop(0, 80, zfill, 0)
    pltpu.sync_copy(zb, den_sh.at[pl.ds(s * DR_PT, DR_PT)])

    pltpu.sync_copy(par_hbm, parb)
    plsc.subcore_barrier()

    cvec = parb[pl.ds(0, L)]
    mvec = parb[pl.ds(L, L)]
    zeros_i = jnp.zeros((L,), jnp.int32)
    one_i = zeros_i + 1

    def chunk(i, _):
        row = (c * NSUB + s) * WCH_PT + i
        pltpu.sync_copy(edata_hbm.at[row], srcv)
        pltpu.sync_copy(edata_hbm.at[ERWS + row], trgv)
        pltpu.sync_copy(edata_hbm.at[2 * ERWS + row], probv)
        d1 = pltpu.async_copy(sctab_hbm.at[srcv], ssrcb, sem)
        d2 = pltpu.async_copy(sctab_hbm.at[trgv], strgb, sem2)
        d1.wait()
        d2.wait()
        for k in range(C // L):
            tvec = trgv[pl.ds(k * L, L)]
            didxv[pl.ds(k * L, L)] = lax.div(tvec, 8)

        def ebody(e, _):
            g = lax.div(e, L)
            j = lax.rem(e, L)
            jsp = zeros_i + j
            pvec = probv[pl.ds(g * L, L)]
            psp = pvec.at[jsp].get(mode="promise_in_bounds").astype(F32)
            sv = (ssrcb[e, pl.ds(0, L)] + strgb[e, pl.ds(L, L)]
                  + cvec * psp)
            sv = jnp.maximum(sv, 0.2 * sv) - mvec
            w = jnp.exp(sv)
            wb[lax.div(e, 8), pl.ds(lax.rem(e, 8) * L, L)] = w
            tvec = trgv[pl.ds(g * L, L)]
            tsp = tvec.at[jsp].get(mode="promise_in_bounds")
            msp = lax.rem(tsp, 8)
            for t in range(8):
                eqt = one_i - jnp.minimum(jnp.abs(msp - t), one_i)
                denb[e, pl.ds(t * L, L)] = w * eqt.astype(F32)
            return 0
        lax.fori_loop(0, C, ebody, 0)

        pltpu.sync_copy(denb, den_sh.at[didxv], add=True)
        pltpu.sync_copy(wb, w_out.at[pl.ds(row * (C // 8), C // 8)])
        return 0

    lax.fori_loop(0, WCH_PT, chunk, 0)
    plsc.subcore_barrier()

    pltpu.sync_copy(den_sh.at[pl.ds(s * DR_PT, DR_PT)],
                    den_out.at[pl.ds(c * DROWS + s * DR_PT, DR_PT)])


_SC_W_CACHE = {}


def _sc_w(sctab, edata, par):
    if "w" in _SC_W_CACHE:
        return _SC_W_CACHE["w"](sctab, edata, par)
    mesh = plsc.VectorSubcoreMesh(core_axis_name="c", subcore_axis_name="s")
    f = pl.kernel(
        _sc_w_body,
        out_type=[jax.ShapeDtypeStruct((WROWS, 128), F32),
                  jax.ShapeDtypeStruct((2 * DROWS, 128), F32)],
        mesh=mesh,
        scratch_types=[
            pltpu.VMEM((C,), jnp.int32),
            pltpu.VMEM((C,), jnp.int32),
            pltpu.VMEM((C,), jnp.int32),
            pltpu.VMEM((C,), jnp.int32),
            pltpu.VMEM((C, 128), F32),
            pltpu.VMEM((C, 128), F32),
            pltpu.VMEM((C // 8, 128), F32),
            pltpu.VMEM((C, 128), F32),
            pltpu.VMEM((128,), F32),
            pltpu.VMEM((80, 128), F32),
            pltpu.VMEM_SHARED((DROWS, 128), F32),
            pltpu.SemaphoreType.DMA,
            pltpu.SemaphoreType.DMA,
        ],
    )
    _SC_W_CACHE["w"] = f
    return f(sctab, edata, par)


def _sc_n_body(proj2_hbm, edata_hbm, w_hbm, hpar_hbm, num_out,
               srcv, trgv, rowv, projb, wbuf, numb, zb, hparb, acc_sh, sem):
    c = lax.axis_index("c")
    s = lax.axis_index("s")

    zrow = jnp.zeros((L,), F32)
    def zfill(i, _):
        for k in range(8):
            zb[i, pl.ds(k * L, L)] = zrow
        return 0
    lax.fori_loop(0, 80, zfill, 0)
    pltpu.sync_copy(hpar_hbm, hparb)

    off = c * N
    zeros_i = jnp.zeros((L,), jnp.int32)
    one_i = zeros_i + 1
    fvec = hparb[pl.ds(0, L)]
    gvec = hparb[pl.ds(L, L)]
    hidx = [c * fvec + k * gvec for k in range(8)]

    for p in range(NPH):
        for zo in range(0, PH_PT, 80):
            pltpu.sync_copy(zb, acc_sh.at[pl.ds(s * PH_PT + zo, 80)])
        plsc.subcore_barrier()

        def chunk(i, _):
            row = s * NCHUNK + i
            pltpu.sync_copy(edata_hbm.at[row], srcv)
            pltpu.sync_copy(edata_hbm.at[ERWS + row], trgv)
            pltpu.sync_copy(w_hbm.at[pl.ds(row * (C // 8), C // 8)], wbuf)
            for k in range(C // L):
                tvec = trgv[pl.ds(k * L, L)]
                q = tvec - p * PH
                a = jnp.minimum(jnp.maximum(q, 0), PH - 1)
                eq = one_i - jnp.minimum(jnp.abs(q - a), one_i)
                rowv[pl.ds(k * L, L)] = DUMP + (a - DUMP) * eq
                srcv[pl.ds(k * L, L)] = srcv[pl.ds(k * L, L)] + off
            pltpu.async_copy(proj2_hbm.at[srcv], projb, sem).wait()

            def ebody(e, _):
                w = wbuf[lax.div(e, 8), pl.ds(lax.rem(e, 8) * L, L)]
                for k in range(8):
                    pv = projb[e, pl.ds(k * L, L)]
                    wk = w.at[hidx[k]].get(mode="promise_in_bounds")
                    numb[e, pl.ds(k * L, L)] = pv * wk
                return 0
            lax.fori_loop(0, C, ebody, 0)

            pltpu.sync_copy(numb, acc_sh.at[rowv], add=True)
            return 0

        lax.fori_loop(0, NCHUNK, chunk, 0)
        plsc.subcore_barrier()

        pltpu.sync_copy(acc_sh.at[pl.ds(s * PH_PT, PH_PT)],
                        num_out.at[pl.ds(c * NP2 + p * PH + s * PH_PT,
                                         PH_PT)])
        plsc.subcore_barrier()


def _sc_n(proj2, edata, warr, hpar):
    if "n" in _SC_W_CACHE:
        return _SC_W_CACHE["n"](proj2, edata, warr, hpar)
    mesh = plsc.VectorSubcoreMesh(core_axis_name="c", subcore_axis_name="s")
    f = pl.kernel(
        _sc_n_body,
        out_type=jax.ShapeDtypeStruct((2 * NP2, 128), F32),
        mesh=mesh,
        scratch_types=[
            pltpu.VMEM((C,), jnp.int32),
            pltpu.VMEM((C,), jnp.int32),
            pltpu.VMEM((C,), jnp.int32),
            pltpu.VMEM((C, 128), F32),
            pltpu.VMEM((C // 8, 128), F32),
            pltpu.VMEM((C, 128), F32),
            pltpu.VMEM((80, 128), F32),
            pltpu.VMEM((128,), jnp.int32),
            pltpu.VMEM_SHARED((AROWS, 128), F32),
            pltpu.SemaphoreType.DMA,
        ],
    )
    _SC_W_CACHE["n"] = f
    return f(proj2, edata, warr, hpar)


def _sc_gather_body(h_hbm, idx_hbm, out_hbm, idxv, rowsb, sem):
    c = lax.axis_index("c")
    s = lax.axis_index("s")
    wid = s * NCORE + c
    for j in range(2):
        base = wid * 2 + j
        pltpu.sync_copy(idx_hbm.at[base], idxv)
        pltpu.async_copy(h_hbm.at[idxv], rowsb, sem).wait()
        pltpu.sync_copy(rowsb, out_hbm.at[pl.ds(base * 128, 128)])


def _sc_gather(h, idx):
    nidx = idx.shape[0]
    idx = idx.reshape(nidx // 128, 128)
    mesh = plsc.VectorSubcoreMesh(core_axis_name="c", subcore_axis_name="s")
    f = pl.kernel(
        _sc_gather_body,
        out_type=jax.ShapeDtypeStruct((nidx, D), F32),
        mesh=mesh,
        scratch_types=[
            pltpu.VMEM((128,), jnp.int32),
            pltpu.VMEM((128, D), F32),
            pltpu.SemaphoreType.DMA,
        ],
    )
    return f(h, idx)


def _layer(h, edata, p, concat, nh):
    fout = D // nh
    W = p['W']
    ssrc_flat = p['scoring_src'].reshape(-1)
    strg_flat = p['scoring_trg'].reshape(-1)
    S = np.zeros((D, L), np.float32)
    for j in range(D):
        S[j, j // fout] = 1.0
    S = jnp.asarray(S)
    Wsrc = (W * ssrc_flat[None, :]) @ S
    Wtrg = (W * strg_flat[None, :]) @ S
    Wskip = p['W_skip'] if 'W_skip' in p else jnp.eye(h.shape[1], D, dtype=F32)

    proj2, sctab, skip, smax = _tc_pre(h, W, Wsrc, Wtrg, Wskip)

    cvec = (p['W_prob'].reshape(nh, fout) * p['scoring_prob'][0]).sum(-1)
    c16 = jnp.zeros((L,), F32).at[:nh].set(cvec)
    mh = smax[0] + smax[1] + jnp.maximum(c16, 0.0)
    mh = jnp.maximum(mh, 0.2 * mh)
    M = jnp.max(mh[:nh])
    c16 = c16 * (1.0 / 16777216.0)
    par = jnp.concatenate([c16, jnp.full((L,), M, F32),
                           jnp.zeros((128 - 2 * L,), F32)])

    warr, den2 = _sc_w(sctab, edata, par)
    f_h = 8 if nh == 16 else 0
    g_h = 1 if nh == 16 else 0
    hpar = jnp.concatenate([jnp.full((L,), f_h, jnp.int32),
                            jnp.full((L,), g_h, jnp.int32),
                            jnp.zeros((128 - 2 * L,), jnp.int32)])
    num2 = _sc_n(proj2.reshape(2 * N, 128), edata, warr, hpar)

    den2 = den2.reshape(2, NP, L)

    R = np.zeros((L, D), np.float32)
    for j in range(D):
        R[j // fout, j] = 1.0
    R = jnp.asarray(R)
    ln = 'ln_gamma' in p
    gamma = p['ln_gamma'].reshape(1, D) if ln else jnp.zeros((1, D), F32)
    beta = p['ln_beta'].reshape(1, D) if ln else jnp.zeros((1, D), F32)
    return _tc_post(num2.reshape(2, NP2, 128), den2[0], den2[1], skip, R,
                    p['bias'].reshape(1, D), gamma, beta, ln)


def kernel(node_features, edge_index_input, edge_prob_input, x, params):
    npad = EPAD - E
    src = jnp.concatenate([edge_index_input[0].astype(jnp.int32),
                           jnp.zeros((npad,), jnp.int32)])
    trg = jnp.concatenate([edge_index_input[1].astype(jnp.int32),
                           jnp.full((npad,), NP - 1, jnp.int32)])
    prob = jnp.concatenate([edge_prob_input[:, 0],
                            jnp.zeros((npad,), F32)])
    pfix = (prob * 16777216.0).astype(jnp.int32)
    edata = jnp.concatenate([
        src.reshape(ERWS, C), trg.reshape(ERWS, C), pfix.reshape(ERWS, C),
        jnp.zeros((EDROWS - 3 * ERWS, C), jnp.int32)])

    h = _layer(node_features, edata, params['enc'], True, 16)
    h = _layer(h, edata, params['gm0'], True, 16)
    h = _layer(h, edata, params['gm1'], False, 1)

    B, S_ = x.shape
    idx = x.reshape(-1).astype(jnp.int32)
    out = _sc_gather(h, idx)
    return out.reshape(B, S_, D)


# speedup vs baseline: 11.3887x; 1.0042x over previous
import functools

import jax
import jax.numpy as jnp
import numpy as np
from jax import lax
from jax.experimental import pallas as pl
from jax.experimental.pallas import tpu as pltpu
from jax.experimental.pallas import tpu_sc as plsc

N = 10000
E = 320000
D = 256
L = 16
NSUB = 16
NCORE = 2
BN = 400
C = 128
EPAD = 327680
ERWS = EPAD // C
EDROWS = 10496
NP = 10240
F32 = jnp.float32

WCH_PT = ERWS // NCORE // NSUB
DROWS = NP // 8
DR_PT = DROWS // NSUB
WROWS = EPAD // 8

PH = 10240
NPH = 1
NP2 = PH * NPH
DUMP = PH
AROWS = PH + 8
PH_PT = PH // NSUB
NCHUNK = ERWS // NSUB


def _tc_pre_body(h_ref, w_ref, wsrc_ref, wtrg_ref, wskip_ref,
                 proj2_ref, sctab_ref, skip_ref, smax_ref):
    i = pl.program_id(0)
    h = h_ref[...]
    proj = jnp.dot(h, w_ref[...], preferred_element_type=F32)
    ssrc = jnp.dot(h, wsrc_ref[...], preferred_element_type=F32)
    strg = jnp.dot(h, wtrg_ref[...], preferred_element_type=F32)
    skip_ref[...] = jnp.dot(h, wskip_ref[...], preferred_element_type=F32)
    proj2_ref[0] = proj[:, :128]
    proj2_ref[1] = proj[:, 128:]
    sctab_ref[...] = jnp.concatenate(
        [ssrc, strg, jnp.zeros((ssrc.shape[0], 128 - 2 * L), F32)], axis=1)

    @pl.when(i == 0)
    def _():
        smax_ref[...] = jnp.full((8, L), -1e30, F32)

    upd = jnp.concatenate(
        [jnp.max(ssrc, axis=0, keepdims=True),
         jnp.max(strg, axis=0, keepdims=True),
         jnp.full((6, L), -1e30, F32)], axis=0)
    smax_ref[...] = jnp.maximum(smax_ref[...], upd)


def _tc_pre(h, W, Wsrc, Wtrg, Wskip):
    fin = h.shape[1]
    nb = N // BN
    out_shapes = [
        jax.ShapeDtypeStruct((2, N, 128), F32),
        jax.ShapeDtypeStruct((N, 128), F32),
        jax.ShapeDtypeStruct((N, D), F32),
        jax.ShapeDtypeStruct((8, L), F32),
    ]
    grid_spec = pl.GridSpec(
        grid=(nb,),
        in_specs=[
            pl.BlockSpec((BN, fin), lambda i: (i, 0)),
            pl.BlockSpec((fin, D), lambda i: (0, 0)),
            pl.BlockSpec((fin, L), lambda i: (0, 0)),
            pl.BlockSpec((fin, L), lambda i: (0, 0)),
            pl.BlockSpec((fin, D), lambda i: (0, 0)),
        ],
        out_specs=[
            pl.BlockSpec((2, BN, 128), lambda i: (0, i, 0)),
            pl.BlockSpec((BN, 128), lambda i: (i, 0)),
            pl.BlockSpec((BN, D), lambda i: (i, 0)),
            pl.BlockSpec((8, L), lambda i: (0, 0)),
        ],
    )
    return pl.pallas_call(_tc_pre_body, grid_spec=grid_spec,
                          out_shape=out_shapes)(h, W, Wsrc, Wtrg, Wskip)


def _tc_post_body(ln, num2_ref, den0_ref, den1_ref, skip_ref, r_ref,
                  bias_ref, gamma_ref, beta_ref, out_ref):
    num = jnp.concatenate([num2_ref[0], num2_ref[1]], axis=1)
    den = den0_ref[...] + den1_ref[...]
    den_exp = jnp.dot(den, r_ref[...], preferred_element_type=F32)
    o = num / (den_exp + 1e-16) + skip_ref[...] + bias_ref[...]
    o = jnp.where(o > 0, o, jnp.exp(jnp.minimum(o, 0.0)) - 1.0)
    if ln:
        mu = jnp.mean(o, axis=1, keepdims=True)
        var = jnp.mean((o - mu) ** 2, axis=1, keepdims=True)
        o = gamma_ref[...] * (o - mu) / jnp.sqrt(var + 1e-5) + beta_ref[...]
    out_ref[...] = o


def _tc_post(num2, den0, den1, skip, R, bias, gamma, beta, ln):
    nb = N // BN
    grid_spec = pl.GridSpec(
        grid=(nb,),
        in_specs=[
            pl.BlockSpec((2, BN, 128), lambda i: (0, i, 0)),
            pl.BlockSpec((BN, L), lambda i: (i, 0)),
            pl.BlockSpec((BN, L), lambda i: (i, 0)),
            pl.BlockSpec((BN, D), lambda i: (i, 0)),
            pl.BlockSpec((L, D), lambda i: (0, 0)),
            pl.BlockSpec((1, D), lambda i: (0, 0)),
            pl.BlockSpec((1, D), lambda i: (0, 0)),
            pl.BlockSpec((1, D), lambda i: (0, 0)),
        ],
        out_specs=pl.BlockSpec((BN, D), lambda i: (i, 0)),
    )
    return pl.pallas_call(functools.partial(_tc_post_body, ln),
                          grid_spec=grid_spec,
                          out_shape=jax.ShapeDtypeStruct((N, D), F32))(
        num2, den0, den1, skip, R, bias, gamma, beta)


def _sc_w_body(sctab_hbm, edata_hbm, par_hbm, w_out, den_out,
               srcv, trgv, didxv, probv, ssrcb, strgb, wb, denb,
               parb, zb, den_sh, sem, sem2):
    c = lax.axis_index("c")
    s = lax.axis_index("s")

    zrow = jnp.zeros((L,), F32)
    def zfill(i, _):
        for k in range(8):
            zb[i, pl.ds(k * L, L)] = zrow
        return 0
    lax.fori_loop(0, 80, zfill, 0)
    pltpu.sync_copy(zb, den_sh.at[pl.ds(s * DR_PT, DR_PT)])

    pltpu.sync_copy(par_hbm, parb)
    plsc.subcore_barrier()

    cvec = parb[pl.ds(0, L)]
    mvec = parb[pl.ds(L, L)]
    zeros_i = jnp.zeros((L,), jnp.int32)
    one_i = zeros_i + 1

    def chunk(i, _):
        row = (c * NSUB + s) * WCH_PT + i
        pltpu.sync_copy(edata_hbm.at[row], srcv)
        pltpu.sync_copy(edata_hbm.at[ERWS + row], trgv)
        pltpu.sync_copy(edata_hbm.at[2 * ERWS + row], probv)
        d1 = pltpu.async_copy(sctab_hbm.at[srcv], ssrcb, sem)
        d2 = pltpu.async_copy(sctab_hbm.at[trgv], strgb, sem2)
        d1.wait()
        d2.wait()
        for k in range(C // L):
            tvec = trgv[pl.ds(k * L, L)]
            didxv[pl.ds(k * L, L)] = lax.div(tvec, 8)

        def ebody(e, _):
            g = lax.div(e, L)
            j = lax.rem(e, L)
            jsp = zeros_i + j
            pvec = probv[pl.ds(g * L, L)]
            psp = pvec.at[jsp].get(mode="promise_in_bounds").astype(F32)
            sv = (ssrcb[e, pl.ds(0, L)] + strgb[e, pl.ds(L, L)]
                  + cvec * psp)
            sv = jnp.maximum(sv, 0.2 * sv) - mvec
            w = jnp.exp(sv)
            wb[lax.div(e, 8), pl.ds(lax.rem(e, 8) * L, L)] = w
            tvec = trgv[pl.ds(g * L, L)]
            tsp = tvec.at[jsp].get(mode="promise_in_bounds")
            msp = lax.rem(tsp, 8)
            for t in range(8):
                eqt = one_i - jnp.minimum(jnp.abs(msp - t), one_i)
                denb[e, pl.ds(t * L, L)] = w * eqt.astype(F32)
            return 0
        lax.fori_loop(0, C, ebody, 0)

        pltpu.sync_copy(denb, den_sh.at[didxv], add=True)
        pltpu.sync_copy(wb, w_out.at[pl.ds(row * (C // 8), C // 8)])
        return 0

    lax.fori_loop(0, WCH_PT, chunk, 0)
    plsc.subcore_barrier()

    pltpu.sync_copy(den_sh.at[pl.ds(s * DR_PT, DR_PT)],
                    den_out.at[pl.ds(c * DROWS + s * DR_PT, DR_PT)])


_SC_W_CACHE = {}


def _sc_w(sctab, edata, par):
    if "w" in _SC_W_CACHE:
        return _SC_W_CACHE["w"](sctab, edata, par)
    mesh = plsc.VectorSubcoreMesh(core_axis_name="c", subcore_axis_name="s")
    f = pl.kernel(
        _sc_w_body,
        out_type=[jax.ShapeDtypeStruct((WROWS, 128), F32),
                  jax.ShapeDtypeStruct((2 * DROWS, 128), F32)],
        mesh=mesh,
        scratch_types=[
            pltpu.VMEM((C,), jnp.int32),
            pltpu.VMEM((C,), jnp.int32),
            pltpu.VMEM((C,), jnp.int32),
            pltpu.VMEM((C,), jnp.int32),
            pltpu.VMEM((C, 128), F32),
            pltpu.VMEM((C, 128), F32),
            pltpu.VMEM((C // 8, 128), F32),
            pltpu.VMEM((C, 128), F32),
            pltpu.VMEM((128,), F32),
            pltpu.VMEM((80, 128), F32),
            pltpu.VMEM_SHARED((DROWS, 128), F32),
            pltpu.SemaphoreType.DMA,
            pltpu.SemaphoreType.DMA,
        ],
    )
    _SC_W_CACHE["w"] = f
    return f(sctab, edata, par)


def _sc_n_body(proj2_hbm, edata_hbm, w_hbm, hpar_hbm, num_out,
               srcv, trgv, wbuf, projb, numb, zb, hparb, acc_sh,
               semG0, semG1):
    c = lax.axis_index("c")
    s = lax.axis_index("s")
    H = C // 2

    zrow = jnp.zeros((L,), F32)
    def zfill(i, _):
        for k in range(8):
            zb[i, pl.ds(k * L, L)] = zrow
        return 0
    lax.fori_loop(0, 80, zfill, 0)
    pltpu.sync_copy(hpar_hbm, hparb)

    off = c * N
    fvec = hparb[pl.ds(0, L)]
    gvec = hparb[pl.ds(L, L)]
    hidx = [c * fvec + k * gvec for k in range(8)]

    for zo in range(0, PH_PT, 80):
        pltpu.sync_copy(zb, acc_sh.at[pl.ds(s * PH_PT + zo, 80)])
    plsc.subcore_barrier()

    def ebody(e, _):
        w = wbuf[lax.div(e, 8), pl.ds(lax.rem(e, 8) * L, L)]
        for k in range(8):
            pv = projb[e, pl.ds(k * L, L)]
            wk = w.at[hidx[k]].get(mode="promise_in_bounds")
            numb[e, pl.ds(k * L, L)] = pv * wk
        return 0

    def chunk(i, _):
        row = s * NCHUNK + i
        pltpu.sync_copy(edata_hbm.at[row], srcv)
        pltpu.sync_copy(edata_hbm.at[ERWS + row], trgv)
        pltpu.sync_copy(w_hbm.at[pl.ds(row * (C // 8), C // 8)], wbuf)
        for k in range(C // L):
            srcv[pl.ds(k * L, L)] = srcv[pl.ds(k * L, L)] + off
        d0 = pltpu.async_copy(proj2_hbm.at[srcv.at[pl.ds(0, H)]],
                              projb.at[pl.ds(0, H)], semG0)
        d1 = pltpu.async_copy(proj2_hbm.at[srcv.at[pl.ds(H, H)]],
                              projb.at[pl.ds(H, H)], semG1)
        d0.wait()
        lax.fori_loop(0, H, ebody, 0)
        d1.wait()
        lax.fori_loop(H, C, ebody, 0)
        pltpu.sync_copy(numb, acc_sh.at[trgv], add=True)
        return 0
    lax.fori_loop(0, NCHUNK, chunk, 0)

    plsc.subcore_barrier()
    pltpu.sync_copy(acc_sh.at[pl.ds(s * PH_PT, PH_PT)],
                    num_out.at[pl.ds(c * NP2 + s * PH_PT, PH_PT)])


def _sc_n(proj2, edata, warr, hpar):
    if "n" in _SC_W_CACHE:
        return _SC_W_CACHE["n"](proj2, edata, warr, hpar)
    mesh = plsc.VectorSubcoreMesh(core_axis_name="c", subcore_axis_name="s")
    f = pl.kernel(
        _sc_n_body,
        out_type=jax.ShapeDtypeStruct((2 * NP2, 128), F32),
        mesh=mesh,
        scratch_types=[
            pltpu.VMEM((C,), jnp.int32),
            pltpu.VMEM((C,), jnp.int32),
            pltpu.VMEM((C // 8, 128), F32),
            pltpu.VMEM((C, 128), F32),
            pltpu.VMEM((C, 128), F32),
            pltpu.VMEM((80, 128), F32),
            pltpu.VMEM((128,), jnp.int32),
            pltpu.VMEM_SHARED((AROWS, 128), F32),
            pltpu.SemaphoreType.DMA,
            pltpu.SemaphoreType.DMA,
        ],
    )
    _SC_W_CACHE["n"] = f
    return f(proj2, edata, warr, hpar)


def _sc_gather_body(h_hbm, idx_hbm, out_hbm, idxv, rowsb, sem):
    c = lax.axis_index("c")
    s = lax.axis_index("s")
    wid = s * NCORE + c
    for j in range(2):
        base = wid * 2 + j
        pltpu.sync_copy(idx_hbm.at[base], idxv)
        pltpu.async_copy(h_hbm.at[idxv], rowsb, sem).wait()
        pltpu.sync_copy(rowsb, out_hbm.at[pl.ds(base * 128, 128)])


def _sc_gather(h, idx):
    nidx = idx.shape[0]
    idx = idx.reshape(nidx // 128, 128)
    mesh = plsc.VectorSubcoreMesh(core_axis_name="c", subcore_axis_name="s")
    f = pl.kernel(
        _sc_gather_body,
        out_type=jax.ShapeDtypeStruct((nidx, D), F32),
        mesh=mesh,
        scratch_types=[
            pltpu.VMEM((128,), jnp.int32),
            pltpu.VMEM((128, D), F32),
            pltpu.SemaphoreType.DMA,
        ],
    )
    return f(h, idx)


def _layer(h, edata, p, concat, nh):
    fout = D // nh
    W = p['W']
    ssrc_flat = p['scoring_src'].reshape(-1)
    strg_flat = p['scoring_trg'].reshape(-1)
    S = np.zeros((D, L), np.float32)
    for j in range(D):
        S[j, j // fout] = 1.0
    S = jnp.asarray(S)
    Wsrc = (W * ssrc_flat[None, :]) @ S
    Wtrg = (W * strg_flat[None, :]) @ S
    Wskip = p['W_skip'] if 'W_skip' in p else jnp.eye(h.shape[1], D, dtype=F32)

    proj2, sctab, skip, smax = _tc_pre(h, W, Wsrc, Wtrg, Wskip)

    cvec = (p['W_prob'].reshape(nh, fout) * p['scoring_prob'][0]).sum(-1)
    c16 = jnp.zeros((L,), F32).at[:nh].set(cvec)
    mh = smax[0] + smax[1] + jnp.maximum(c16, 0.0)
    mh = jnp.maximum(mh, 0.2 * mh)
    M = jnp.max(mh[:nh])
    c16 = c16 * (1.0 / 16777216.0)
    par = jnp.concatenate([c16, jnp.full((L,), M, F32),
                           jnp.zeros((128 - 2 * L,), F32)])

    warr, den2 = _sc_w(sctab, edata, par)
    f_h = 8 if nh == 16 else 0
    g_h = 1 if nh == 16 else 0
    hpar = jnp.concatenate([jnp.full((L,), f_h, jnp.int32),
                            jnp.full((L,), g_h, jnp.int32),
                            jnp.zeros((128 - 2 * L,), jnp.int32)])
    num2 = _sc_n(proj2.reshape(2 * N, 128), edata, warr, hpar)

    den2 = den2.reshape(2, NP, L)

    R = np.zeros((L, D), np.float32)
    for j in range(D):
        R[j // fout, j] = 1.0
    R = jnp.asarray(R)
    ln = 'ln_gamma' in p
    gamma = p['ln_gamma'].reshape(1, D) if ln else jnp.zeros((1, D), F32)
    beta = p['ln_beta'].reshape(1, D) if ln else jnp.zeros((1, D), F32)
    return _tc_post(num2.reshape(2, NP2, 128), den2[0], den2[1], skip, R,
                    p['bias'].reshape(1, D), gamma, beta, ln)


def kernel(node_features, edge_index_input, edge_prob_input, x, params):
    npad = EPAD - E
    src = jnp.concatenate([edge_index_input[0].astype(jnp.int32),
                           jnp.zeros((npad,), jnp.int32)])
    trg = jnp.concatenate([edge_index_input[1].astype(jnp.int32),
                           jnp.full((npad,), NP - 1, jnp.int32)])
    prob = jnp.concatenate([edge_prob_input[:, 0],
                            jnp.zeros((npad,), F32)])
    pfix = (prob * 16777216.0).astype(jnp.int32)
    edata = jnp.concatenate([
        src.reshape(ERWS, C), trg.reshape(ERWS, C), pfix.reshape(ERWS, C),
        jnp.zeros((EDROWS - 3 * ERWS, C), jnp.int32)])

    h = _layer(node_features, edata, params['enc'], True, 16)
    h = _layer(h, edata, params['gm0'], True, 16)
    h = _layer(h, edata, params['gm1'], False, 1)

    B, S_ = x.shape
    idx = x.reshape(-1).astype(jnp.int32)
    out = _sc_gather(h, idx)
    return out.reshape(B, S_, D)


# speedup vs baseline: 11.8114x; 1.0371x over previous
import functools

import jax
import jax.numpy as jnp
import numpy as np
from jax import lax
from jax.experimental import pallas as pl
from jax.experimental.pallas import tpu as pltpu
from jax.experimental.pallas import tpu_sc as plsc

N = 10000
E = 320000
D = 256
L = 16
NSUB = 16
NCORE = 2
BN = 400
C = 128
EPAD = 327680
ERWS = EPAD // C
EDROWS = 10496
NP = 10240
F32 = jnp.float32

WCH_PT = ERWS // NCORE // NSUB
DROWS = NP // 8
DR_PT = DROWS // NSUB
WROWS = EPAD // 8

PH = 10240
NPH = 1
NP2 = PH * NPH
DUMP = PH
AROWS = PH + 8
PH_PT = PH // NSUB
NCHUNK = ERWS // NSUB


def _tc_pre_body(h_ref, w_ref, wsrc_ref, wtrg_ref, wskip_ref,
                 proj2_ref, sctab_ref, skip_ref, smax_ref):
    i = pl.program_id(0)
    h = h_ref[...]
    proj = jnp.dot(h, w_ref[...], preferred_element_type=F32)
    ssrc = jnp.dot(h, wsrc_ref[...], preferred_element_type=F32)
    strg = jnp.dot(h, wtrg_ref[...], preferred_element_type=F32)
    skip_ref[...] = jnp.dot(h, wskip_ref[...], preferred_element_type=F32)
    proj2_ref[0] = proj[:, :128]
    proj2_ref[1] = proj[:, 128:]
    sctab_ref[...] = jnp.concatenate(
        [ssrc, strg, jnp.zeros((ssrc.shape[0], 128 - 2 * L), F32)], axis=1)

    @pl.when(i == 0)
    def _():
        smax_ref[...] = jnp.full((8, L), -1e30, F32)

    upd = jnp.concatenate(
        [jnp.max(ssrc, axis=0, keepdims=True),
         jnp.max(strg, axis=0, keepdims=True),
         jnp.full((6, L), -1e30, F32)], axis=0)
    smax_ref[...] = jnp.maximum(smax_ref[...], upd)


def _tc_pre(h, W, Wsrc, Wtrg, Wskip):
    fin = h.shape[1]
    nb = N // BN
    out_shapes = [
        jax.ShapeDtypeStruct((2, N, 128), F32),
        jax.ShapeDtypeStruct((N, 128), F32),
        jax.ShapeDtypeStruct((N, D), F32),
        jax.ShapeDtypeStruct((8, L), F32),
    ]
    grid_spec = pl.GridSpec(
        grid=(nb,),
        in_specs=[
            pl.BlockSpec((BN, fin), lambda i: (i, 0)),
            pl.BlockSpec((fin, D), lambda i: (0, 0)),
            pl.BlockSpec((fin, L), lambda i: (0, 0)),
            pl.BlockSpec((fin, L), lambda i: (0, 0)),
            pl.BlockSpec((fin, D), lambda i: (0, 0)),
        ],
        out_specs=[
            pl.BlockSpec((2, BN, 128), lambda i: (0, i, 0)),
            pl.BlockSpec((BN, 128), lambda i: (i, 0)),
            pl.BlockSpec((BN, D), lambda i: (i, 0)),
            pl.BlockSpec((8, L), lambda i: (0, 0)),
        ],
    )
    return pl.pallas_call(_tc_pre_body, grid_spec=grid_spec,
                          out_shape=out_shapes)(h, W, Wsrc, Wtrg, Wskip)


def _tc_post_body(ln, num2_ref, den0_ref, den1_ref, skip_ref, r_ref,
                  bias_ref, gamma_ref, beta_ref, out_ref):
    num = jnp.concatenate([num2_ref[0], num2_ref[1]], axis=1)
    den = den0_ref[...] + den1_ref[...]
    den_exp = jnp.dot(den, r_ref[...], preferred_element_type=F32)
    o = num / (den_exp + 1e-16) + skip_ref[...] + bias_ref[...]
    o = jnp.where(o > 0, o, jnp.exp(jnp.minimum(o, 0.0)) - 1.0)
    if ln:
        mu = jnp.mean(o, axis=1, keepdims=True)
        var = jnp.mean((o - mu) ** 2, axis=1, keepdims=True)
        o = gamma_ref[...] * (o - mu) / jnp.sqrt(var + 1e-5) + beta_ref[...]
    out_ref[...] = o


def _tc_post(num2, den0, den1, skip, R, bias, gamma, beta, ln):
    nb = N // BN
    grid_spec = pl.GridSpec(
        grid=(nb,),
        in_specs=[
            pl.BlockSpec((2, BN, 128), lambda i: (0, i, 0)),
            pl.BlockSpec((BN, L), lambda i: (i, 0)),
            pl.BlockSpec((BN, L), lambda i: (i, 0)),
            pl.BlockSpec((BN, D), lambda i: (i, 0)),
            pl.BlockSpec((L, D), lambda i: (0, 0)),
            pl.BlockSpec((1, D), lambda i: (0, 0)),
            pl.BlockSpec((1, D), lambda i: (0, 0)),
            pl.BlockSpec((1, D), lambda i: (0, 0)),
        ],
        out_specs=pl.BlockSpec((BN, D), lambda i: (i, 0)),
    )
    return pl.pallas_call(functools.partial(_tc_post_body, ln),
                          grid_spec=grid_spec,
                          out_shape=jax.ShapeDtypeStruct((N, D), F32))(
        num2, den0, den1, skip, R, bias, gamma, beta)


def _sc_w_body(sctab_hbm, edata_hbm, par_hbm, w_out, den_out,
               srcv, trgv, didxv, probv, ssrcb, strgb, wb, denb,
               parb, zb, den_sh, sem, sem2):
    c = lax.axis_index("c")
    s = lax.axis_index("s")

    zrow = jnp.zeros((L,), F32)
    def zfill(i, _):
        for k in range(8):
            zb[i, pl.ds(k * L, L)] = zrow
        return 0
    lax.fori_loop(0, 80, zfill, 0)
    pltpu.sync_copy(zb, den_sh.at[pl.ds(s * DR_PT, DR_PT)])

    pltpu.sync_copy(par_hbm, parb)
    plsc.subcore_barrier()

    cvec = parb[pl.ds(0, L)]
    mvec = parb[pl.ds(L, L)]
    zeros_i = jnp.zeros((L,), jnp.int32)
    one_i = zeros_i + 1

    def chunk(i, _):
        row = (c * NSUB + s) * WCH_PT + i
        pltpu.sync_copy(edata_hbm.at[row], srcv)
        pltpu.sync_copy(edata_hbm.at[ERWS + row], trgv)
        pltpu.sync_copy(edata_hbm.at[2 * ERWS + row], probv)
        d1 = pltpu.async_copy(sctab_hbm.at[srcv], ssrcb, sem)
        d2 = pltpu.async_copy(sctab_hbm.at[trgv], strgb, sem2)
        d1.wait()
        d2.wait()
        for k in range(C // L):
            tvec = trgv[pl.ds(k * L, L)]
            didxv[pl.ds(k * L, L)] = lax.div(tvec, 8)

        def ebody(e, _):
            g = lax.div(e, L)
            j = lax.rem(e, L)
            jsp = zeros_i + j
            pvec = probv[pl.ds(g * L, L)]
            psp = pvec.at[jsp].get(mode="promise_in_bounds").astype(F32)
            sv = (ssrcb[e, pl.ds(0, L)] + strgb[e, pl.ds(L, L)]
                  + cvec * psp)
            sv = jnp.maximum(sv, 0.2 * sv) - mvec
            w = jnp.exp(sv)
            wb[lax.div(e, 8), pl.ds(lax.rem(e, 8) * L, L)] = w
            tvec = trgv[pl.ds(g * L, L)]
            tsp = tvec.at[jsp].get(mode="promise_in_bounds")
            msp = lax.rem(tsp, 8)
            for t in range(8):
                eqt = one_i - jnp.minimum(jnp.abs(msp - t), one_i)
                denb[e, pl.ds(t * L, L)] = w * eqt.astype(F32)
            return 0
        lax.fori_loop(0, C, ebody, 0)

        pltpu.sync_copy(denb, den_sh.at[didxv], add=True)
        pltpu.sync_copy(wb, w_out.at[pl.ds(row * (C // 8), C // 8)])
        return 0

    lax.fori_loop(0, WCH_PT, chunk, 0)
    plsc.subcore_barrier()

    pltpu.sync_copy(den_sh.at[pl.ds(s * DR_PT, DR_PT)],
                    den_out.at[pl.ds(c * DROWS + s * DR_PT, DR_PT)])


_SC_W_CACHE = {}


def _sc_w(sctab, edata, par):
    if "w" in _SC_W_CACHE:
        return _SC_W_CACHE["w"](sctab, edata, par)
    mesh = plsc.VectorSubcoreMesh(core_axis_name="c", subcore_axis_name="s")
    f = pl.kernel(
        _sc_w_body,
        out_type=[jax.ShapeDtypeStruct((WROWS, 128), F32),
                  jax.ShapeDtypeStruct((2 * DROWS, 128), F32)],
        mesh=mesh,
        scratch_types=[
            pltpu.VMEM((C,), jnp.int32),
            pltpu.VMEM((C,), jnp.int32),
            pltpu.VMEM((C,), jnp.int32),
            pltpu.VMEM((C,), jnp.int32),
            pltpu.VMEM((C, 128), F32),
            pltpu.VMEM((C, 128), F32),
            pltpu.VMEM((C // 8, 128), F32),
            pltpu.VMEM((C, 128), F32),
            pltpu.VMEM((128,), F32),
            pltpu.VMEM((80, 128), F32),
            pltpu.VMEM_SHARED((DROWS, 128), F32),
            pltpu.SemaphoreType.DMA,
            pltpu.SemaphoreType.DMA,
        ],
    )
    _SC_W_CACHE["w"] = f
    return f(sctab, edata, par)


def _sc_n_body(proj2_hbm, edata_hbm, w_hbm, hpar_hbm, num_out,
               srcv, trgv, trgva, trgvb, wbuf, projb, numba, numbb,
               zb, hparb, acc_sh, semG0, semG1, semS0, semS1):
    c = lax.axis_index("c")
    s = lax.axis_index("s")
    H = C // 2

    zrow = jnp.zeros((L,), F32)
    def zfill(i, _):
        for k in range(8):
            zb[i, pl.ds(k * L, L)] = zrow
        return 0
    lax.fori_loop(0, 16, zfill, 0)
    pltpu.sync_copy(hpar_hbm, hparb)

    off = c * N
    fvec = hparb[pl.ds(0, L)]
    gvec = hparb[pl.ds(L, L)]
    hidx = [c * fvec + k * gvec for k in range(8)]

    for zo in range(0, PH_PT, 16):
        pltpu.sync_copy(zb, acc_sh.at[pl.ds(s * PH_PT + zo, 16)])
    plsc.subcore_barrier()

    def make_ebody(nb, base):
        def ebody(e, _):
            w = wbuf[lax.div(base + e, 8),
                     pl.ds(lax.rem(base + e, 8) * L, L)]
            for k in range(8):
                pv = projb[base + e, pl.ds(k * L, L)]
                wk = w.at[hidx[k]].get(mode="promise_in_bounds")
                nb[e, pl.ds(k * L, L)] = pv * wk
            return 0
        return ebody

    eb_a = make_ebody(numba, 0)
    eb_b = make_ebody(numbb, H)

    def chunk(i, _):
        row = s * NCHUNK + i
        pltpu.sync_copy(edata_hbm.at[row], srcv)
        pltpu.sync_copy(edata_hbm.at[ERWS + row], trgv)
        pltpu.sync_copy(w_hbm.at[pl.ds(row * (C // 8), C // 8)], wbuf)
        for k in range(C // L):
            srcv[pl.ds(k * L, L)] = srcv[pl.ds(k * L, L)] + off
        d0 = pltpu.async_copy(proj2_hbm.at[srcv.at[pl.ds(0, H)]],
                              projb.at[pl.ds(0, H)], semG0)
        d1 = pltpu.async_copy(proj2_hbm.at[srcv.at[pl.ds(H, H)]],
                              projb.at[pl.ds(H, H)], semG1)

        @pl.when(i > 0)
        def _():
            pltpu.make_async_copy(proj2_hbm.at[pl.ds(0, H)], numba,
                                  semS0).wait()
            pltpu.make_async_copy(proj2_hbm.at[pl.ds(0, H)], numbb,
                                  semS1).wait()
        for k in range(H // L):
            trgva[pl.ds(k * L, L)] = trgv[pl.ds(k * L, L)]
            trgvb[pl.ds(k * L, L)] = trgv[pl.ds(H + k * L, L)]

        d0.wait()
        lax.fori_loop(0, H, eb_a, 0)
        pltpu.async_copy(numba, acc_sh.at[trgva], semS0, add=True)
        d1.wait()
        lax.fori_loop(0, H, eb_b, 0)
        pltpu.async_copy(numbb, acc_sh.at[trgvb], semS1, add=True)
        return 0
    lax.fori_loop(0, NCHUNK, chunk, 0)

    pltpu.make_async_copy(proj2_hbm.at[pl.ds(0, H)], numba, semS0).wait()
    pltpu.make_async_copy(proj2_hbm.at[pl.ds(0, H)], numbb, semS1).wait()
    plsc.subcore_barrier()
    pltpu.sync_copy(acc_sh.at[pl.ds(s * PH_PT, PH_PT)],
                    num_out.at[pl.ds(c * NP2 + s * PH_PT, PH_PT)])


def _sc_n(proj2, edata, warr, hpar):
    if "n" in _SC_W_CACHE:
        return _SC_W_CACHE["n"](proj2, edata, warr, hpar)
    mesh = plsc.VectorSubcoreMesh(core_axis_name="c", subcore_axis_name="s")
    f = pl.kernel(
        _sc_n_body,
        out_type=jax.ShapeDtypeStruct((2 * NP2, 128), F32),
        mesh=mesh,
        scratch_types=[
            pltpu.VMEM((C,), jnp.int32),
            pltpu.VMEM((C,), jnp.int32),
            pltpu.VMEM((C // 2,), jnp.int32),
            pltpu.VMEM((C // 2,), jnp.int32),
            pltpu.VMEM((C // 8, 128), F32),
            pltpu.VMEM((C, 128), F32),
            pltpu.VMEM((C // 2, 128), F32),
            pltpu.VMEM((C // 2, 128), F32),
            pltpu.VMEM((16, 128), F32),
            pltpu.VMEM((128,), jnp.int32),
            pltpu.VMEM_SHARED((AROWS, 128), F32),
            pltpu.SemaphoreType.DMA,
            pltpu.SemaphoreType.DMA,
            pltpu.SemaphoreType.DMA,
            pltpu.SemaphoreType.DMA,
        ],
    )
    _SC_W_CACHE["n"] = f
    return f(proj2, edata, warr, hpar)


def _sc_gather_body(h_hbm, idx_hbm, out_hbm, idxv, rowsb, sem):
    c = lax.axis_index("c")
    s = lax.axis_index("s")
    wid = s * NCORE + c
    for j in range(2):
        base = wid * 2 + j
        pltpu.sync_copy(idx_hbm.at[base], idxv)
        pltpu.async_copy(h_hbm.at[idxv], rowsb, sem).wait()
        pltpu.sync_copy(rowsb, out_hbm.at[pl.ds(base * 128, 128)])


def _sc_gather(h, idx):
    nidx = idx.shape[0]
    idx = idx.reshape(nidx // 128, 128)
    mesh = plsc.VectorSubcoreMesh(core_axis_name="c", subcore_axis_name="s")
    f = pl.kernel(
        _sc_gather_body,
        out_type=jax.ShapeDtypeStruct((nidx, D), F32),
        mesh=mesh,
        scratch_types=[
            pltpu.VMEM((128,), jnp.int32),
            pltpu.VMEM((128, D), F32),
            pltpu.SemaphoreType.DMA,
        ],
    )
    return f(h, idx)


def _layer(h, edata, p, concat, nh):
    fout = D // nh
    W = p['W']
    ssrc_flat = p['scoring_src'].reshape(-1)
    strg_flat = p['scoring_trg'].reshape(-1)
    S = np.zeros((D, L), np.float32)
    for j in range(D):
        S[j, j // fout] = 1.0
    S = jnp.asarray(S)
    Wsrc = (W * ssrc_flat[None, :]) @ S
    Wtrg = (W * strg_flat[None, :]) @ S
    Wskip = p['W_skip'] if 'W_skip' in p else jnp.eye(h.shape[1], D, dtype=F32)

    proj2, sctab, skip, smax = _tc_pre(h, W, Wsrc, Wtrg, Wskip)

    cvec = (p['W_prob'].reshape(nh, fout) * p['scoring_prob'][0]).sum(-1)
    c16 = jnp.zeros((L,), F32).at[:nh].set(cvec)
    mh = smax[0] + smax[1] + jnp.maximum(c16, 0.0)
    mh = jnp.maximum(mh, 0.2 * mh)
    M = jnp.max(mh[:nh])
    c16 = c16 * (1.0 / 16777216.0)
    par = jnp.concatenate([c16, jnp.full((L,), M, F32),
                           jnp.zeros((128 - 2 * L,), F32)])

    warr, den2 = _sc_w(sctab, edata, par)
    f_h = 8 if nh == 16 else 0
    g_h = 1 if nh == 16 else 0
    hpar = jnp.concatenate([jnp.full((L,), f_h, jnp.int32),
                            jnp.full((L,), g_h, jnp.int32),
                            jnp.zeros((128 - 2 * L,), jnp.int32)])
    num2 = _sc_n(proj2.reshape(2 * N, 128), edata, warr, hpar)

    den2 = den2.reshape(2, NP, L)

    R = np.zeros((L, D), np.float32)
    for j in range(D):
        R[j // fout, j] = 1.0
    R = jnp.asarray(R)
    ln = 'ln_gamma' in p
    gamma = p['ln_gamma'].reshape(1, D) if ln else jnp.zeros((1, D), F32)
    beta = p['ln_beta'].reshape(1, D) if ln else jnp.zeros((1, D), F32)
    return _tc_post(num2.reshape(2, NP2, 128), den2[0], den2[1], skip, R,
                    p['bias'].reshape(1, D), gamma, beta, ln)


def kernel(node_features, edge_index_input, edge_prob_input, x, params):
    npad = EPAD - E
    src = jnp.concatenate([edge_index_input[0].astype(jnp.int32),
                           jnp.zeros((npad,), jnp.int32)])
    trg = jnp.concatenate([edge_index_input[1].astype(jnp.int32),
                           jnp.full((npad,), NP - 1, jnp.int32)])
    prob = jnp.concatenate([edge_prob_input[:, 0],
                            jnp.zeros((npad,), F32)])
    pfix = (prob * 16777216.0).astype(jnp.int32)
    edata = jnp.concatenate([
        src.reshape(ERWS, C), trg.reshape(ERWS, C), pfix.reshape(ERWS, C),
        jnp.zeros((EDROWS - 3 * ERWS, C), jnp.int32)])

    h = _layer(node_features, edata, params['enc'], True, 16)
    h = _layer(h, edata, params['gm0'], True, 16)
    h = _layer(h, edata, params['gm1'], False, 1)

    B, S_ = x.shape
    idx = x.reshape(-1).astype(jnp.int32)
    out = _sc_gather(h, idx)
    return out.reshape(B, S_, D)


# speedup vs baseline: 12.0457x; 1.0198x over previous
import functools

import jax
import jax.numpy as jnp
import numpy as np
from jax import lax
from jax.experimental import pallas as pl
from jax.experimental.pallas import tpu as pltpu
from jax.experimental.pallas import tpu_sc as plsc

N = 10000
E = 320000
D = 256
L = 16
NSUB = 16
NCORE = 2
BN = 400
C = 128
EPAD = 327680
ERWS = EPAD // C
EDROWS = 10496
NP = 10240
F32 = jnp.float32

WCH_PT = ERWS // NCORE // NSUB
DROWS = NP // 8
DR_PT = DROWS // NSUB
WROWS = EPAD // 8

PH = 10240
NPH = 1
NP2 = PH * NPH
DUMP = PH
AROWS = PH + 8
PH_PT = PH // NSUB
NCHUNK = ERWS // NSUB


def _tc_pre_body(h_ref, w_ref, wsrc_ref, wtrg_ref, wskip_ref,
                 proj2_ref, sctab_ref, skip_ref, smax_ref):
    i = pl.program_id(0)
    h = h_ref[...]
    proj = jnp.dot(h, w_ref[...], preferred_element_type=F32)
    ssrc = jnp.dot(h, wsrc_ref[...], preferred_element_type=F32)
    strg = jnp.dot(h, wtrg_ref[...], preferred_element_type=F32)
    skip_ref[...] = jnp.dot(h, wskip_ref[...], preferred_element_type=F32)
    proj2_ref[0] = proj[:, :128]
    proj2_ref[1] = proj[:, 128:]
    sctab_ref[...] = jnp.concatenate(
        [ssrc, strg, jnp.zeros((ssrc.shape[0], 128 - 2 * L), F32)], axis=1)

    @pl.when(i == 0)
    def _():
        smax_ref[...] = jnp.full((8, L), -1e30, F32)

    upd = jnp.concatenate(
        [jnp.max(ssrc, axis=0, keepdims=True),
         jnp.max(strg, axis=0, keepdims=True),
         jnp.full((6, L), -1e30, F32)], axis=0)
    smax_ref[...] = jnp.maximum(smax_ref[...], upd)


def _tc_pre(h, W, Wsrc, Wtrg, Wskip):
    fin = h.shape[1]
    nb = N // BN
    out_shapes = [
        jax.ShapeDtypeStruct((2, N, 128), F32),
        jax.ShapeDtypeStruct((N, 128), F32),
        jax.ShapeDtypeStruct((N, D), F32),
        jax.ShapeDtypeStruct((8, L), F32),
    ]
    grid_spec = pl.GridSpec(
        grid=(nb,),
        in_specs=[
            pl.BlockSpec((BN, fin), lambda i: (i, 0)),
            pl.BlockSpec((fin, D), lambda i: (0, 0)),
            pl.BlockSpec((fin, L), lambda i: (0, 0)),
            pl.BlockSpec((fin, L), lambda i: (0, 0)),
            pl.BlockSpec((fin, D), lambda i: (0, 0)),
        ],
        out_specs=[
            pl.BlockSpec((2, BN, 128), lambda i: (0, i, 0)),
            pl.BlockSpec((BN, 128), lambda i: (i, 0)),
            pl.BlockSpec((BN, D), lambda i: (i, 0)),
            pl.BlockSpec((8, L), lambda i: (0, 0)),
        ],
    )
    return pl.pallas_call(_tc_pre_body, grid_spec=grid_spec,
                          out_shape=out_shapes)(h, W, Wsrc, Wtrg, Wskip)


def _tc_post_body(ln, num2_ref, den0_ref, den1_ref, skip_ref, r_ref,
                  bias_ref, gamma_ref, beta_ref, out_ref):
    num = jnp.concatenate([num2_ref[0], num2_ref[1]], axis=1)
    den = den0_ref[...] + den1_ref[...]
    den_exp = jnp.dot(den, r_ref[...], preferred_element_type=F32)
    o = num / (den_exp + 1e-16) + skip_ref[...] + bias_ref[...]
    o = jnp.where(o > 0, o, jnp.exp(jnp.minimum(o, 0.0)) - 1.0)
    if ln:
        mu = jnp.mean(o, axis=1, keepdims=True)
        var = jnp.mean((o - mu) ** 2, axis=1, keepdims=True)
        o = gamma_ref[...] * (o - mu) / jnp.sqrt(var + 1e-5) + beta_ref[...]
    out_ref[...] = o


def _tc_post(num2, den0, den1, skip, R, bias, gamma, beta, ln):
    nb = N // BN
    grid_spec = pl.GridSpec(
        grid=(nb,),
        in_specs=[
            pl.BlockSpec((2, BN, 128), lambda i: (0, i, 0)),
            pl.BlockSpec((BN, L), lambda i: (i, 0)),
            pl.BlockSpec((BN, L), lambda i: (i, 0)),
            pl.BlockSpec((BN, D), lambda i: (i, 0)),
            pl.BlockSpec((L, D), lambda i: (0, 0)),
            pl.BlockSpec((1, D), lambda i: (0, 0)),
            pl.BlockSpec((1, D), lambda i: (0, 0)),
            pl.BlockSpec((1, D), lambda i: (0, 0)),
        ],
        out_specs=pl.BlockSpec((BN, D), lambda i: (i, 0)),
    )
    return pl.pallas_call(functools.partial(_tc_post_body, ln),
                          grid_spec=grid_spec,
                          out_shape=jax.ShapeDtypeStruct((N, D), F32))(
        num2, den0, den1, skip, R, bias, gamma, beta)


def _sc_w_body(sctab_hbm, edata_hbm, par_hbm, w_out, den_out,
               srcv, trgv, didxva, didxvb, probv, ssrcb, strgb, wb,
               denba, denbb, parb, zb, den_sh, sem, sem2, sem3, sem4,
               semD0, semD1):
    c = lax.axis_index("c")
    s = lax.axis_index("s")
    H = C // 2

    zrow = jnp.zeros((L,), F32)
    def zfill(i, _):
        for k in range(8):
            zb[i, pl.ds(k * L, L)] = zrow
        return 0
    lax.fori_loop(0, 80, zfill, 0)
    pltpu.sync_copy(zb, den_sh.at[pl.ds(s * DR_PT, DR_PT)])

    pltpu.sync_copy(par_hbm, parb)
    plsc.subcore_barrier()

    cvec = parb[pl.ds(0, L)]
    mvec = parb[pl.ds(L, L)]
    zeros_i = jnp.zeros((L,), jnp.int32)
    one_i = zeros_i + 1

    def make_ebody(nb, base):
        def ebody(e, _):
            eg = base + e
            g = lax.div(eg, L)
            j = lax.rem(eg, L)
            jsp = zeros_i + j
            pvec = probv[pl.ds(g * L, L)]
            psp = pvec.at[jsp].get(mode="promise_in_bounds").astype(F32)
            sv = (ssrcb[eg, pl.ds(0, L)] + strgb[eg, pl.ds(L, L)]
                  + cvec * psp)
            sv = jnp.maximum(sv, 0.2 * sv) - mvec
            w = jnp.exp(sv)
            wb[lax.div(eg, 8), pl.ds(lax.rem(eg, 8) * L, L)] = w
            tvec = trgv[pl.ds(g * L, L)]
            tsp = tvec.at[jsp].get(mode="promise_in_bounds")
            msp = lax.rem(tsp, 8)
            for t in range(8):
                eqt = one_i - jnp.minimum(jnp.abs(msp - t), one_i)
                nb[e, pl.ds(t * L, L)] = w * eqt.astype(F32)
            return 0
        return ebody

    eb_a = make_ebody(denba, 0)
    eb_b = make_ebody(denbb, H)

    def chunk(i, _):
        row = (c * NSUB + s) * WCH_PT + i
        pltpu.sync_copy(edata_hbm.at[row], srcv)
        pltpu.sync_copy(edata_hbm.at[ERWS + row], trgv)
        pltpu.sync_copy(edata_hbm.at[2 * ERWS + row], probv)
        d1a = pltpu.async_copy(sctab_hbm.at[srcv.at[pl.ds(0, H)]],
                               ssrcb.at[pl.ds(0, H)], sem)
        d2a = pltpu.async_copy(sctab_hbm.at[trgv.at[pl.ds(0, H)]],
                               strgb.at[pl.ds(0, H)], sem2)
        d1b = pltpu.async_copy(sctab_hbm.at[srcv.at[pl.ds(H, H)]],
                               ssrcb.at[pl.ds(H, H)], sem3)
        d2b = pltpu.async_copy(sctab_hbm.at[trgv.at[pl.ds(H, H)]],
                               strgb.at[pl.ds(H, H)], sem4)

        @pl.when(i > 0)
        def _():
            pltpu.make_async_copy(sctab_hbm.at[pl.ds(0, H)], denba,
                                  semD0).wait()
            pltpu.make_async_copy(sctab_hbm.at[pl.ds(0, H)], denbb,
                                  semD1).wait()
        for k in range(C // L):
            tvec = trgv[pl.ds(k * L, L)]
            if k < C // L // 2:
                didxva[pl.ds(k * L, L)] = lax.div(tvec, 8)
            else:
                didxvb[pl.ds(k * L - H, L)] = lax.div(tvec, 8)

        d1a.wait()
        d2a.wait()
        lax.fori_loop(0, H, eb_a, 0)
        pltpu.async_copy(denba, den_sh.at[didxva], semD0, add=True)
        d1b.wait()
        d2b.wait()
        lax.fori_loop(0, H, eb_b, 0)
        pltpu.async_copy(denbb, den_sh.at[didxvb], semD1, add=True)
        pltpu.sync_copy(wb, w_out.at[pl.ds(row * (C // 8), C // 8)])
        return 0

    lax.fori_loop(0, WCH_PT, chunk, 0)
    pltpu.make_async_copy(sctab_hbm.at[pl.ds(0, H)], denba, semD0).wait()
    pltpu.make_async_copy(sctab_hbm.at[pl.ds(0, H)], denbb, semD1).wait()
    plsc.subcore_barrier()

    pltpu.sync_copy(den_sh.at[pl.ds(s * DR_PT, DR_PT)],
                    den_out.at[pl.ds(c * DROWS + s * DR_PT, DR_PT)])


_SC_W_CACHE = {}


def _sc_w(sctab, edata, par):
    if "w" in _SC_W_CACHE:
        return _SC_W_CACHE["w"](sctab, edata, par)
    mesh = plsc.VectorSubcoreMesh(core_axis_name="c", subcore_axis_name="s")
    f = pl.kernel(
        _sc_w_body,
        out_type=[jax.ShapeDtypeStruct((WROWS, 128), F32),
                  jax.ShapeDtypeStruct((2 * DROWS, 128), F32)],
        mesh=mesh,
        scratch_types=[
            pltpu.VMEM((C,), jnp.int32),
            pltpu.VMEM((C,), jnp.int32),
            pltpu.VMEM((C // 2,), jnp.int32),
            pltpu.VMEM((C // 2,), jnp.int32),
            pltpu.VMEM((C,), jnp.int32),
            pltpu.VMEM((C, 128), F32),
            pltpu.VMEM((C, 128), F32),
            pltpu.VMEM((C // 8, 128), F32),
            pltpu.VMEM((C // 2, 128), F32),
            pltpu.VMEM((C // 2, 128), F32),
            pltpu.VMEM((128,), F32),
            pltpu.VMEM((80, 128), F32),
            pltpu.VMEM_SHARED((DROWS, 128), F32),
            pltpu.SemaphoreType.DMA,
            pltpu.SemaphoreType.DMA,
            pltpu.SemaphoreType.DMA,
            pltpu.SemaphoreType.DMA,
            pltpu.SemaphoreType.DMA,
            pltpu.SemaphoreType.DMA,
        ],
    )
    _SC_W_CACHE["w"] = f
    return f(sctab, edata, par)


def _sc_n_body(proj2_hbm, edata_hbm, w_hbm, hpar_hbm, num_out,
               srcv, trgv, trgva, trgvb, wbuf, projb, numba, numbb,
               zb, hparb, acc_sh, semG0, semG1, semS0, semS1):
    c = lax.axis_index("c")
    s = lax.axis_index("s")
    H = C // 2

    zrow = jnp.zeros((L,), F32)
    def zfill(i, _):
        for k in range(8):
            zb[i, pl.ds(k * L, L)] = zrow
        return 0
    lax.fori_loop(0, 16, zfill, 0)
    pltpu.sync_copy(hpar_hbm, hparb)

    off = c * N
    fvec = hparb[pl.ds(0, L)]
    gvec = hparb[pl.ds(L, L)]
    hidx = [c * fvec + k * gvec for k in range(8)]

    for zo in range(0, PH_PT, 16):
        pltpu.sync_copy(zb, acc_sh.at[pl.ds(s * PH_PT + zo, 16)])
    plsc.subcore_barrier()

    def make_ebody(nb, base):
        def ebody(e, _):
            w = wbuf[lax.div(base + e, 8),
                     pl.ds(lax.rem(base + e, 8) * L, L)]
            for k in range(8):
                pv = projb[base + e, pl.ds(k * L, L)]
                wk = w.at[hidx[k]].get(mode="promise_in_bounds")
                nb[e, pl.ds(k * L, L)] = pv * wk
            return 0
        return ebody

    eb_a = make_ebody(numba, 0)
    eb_b = make_ebody(numbb, H)

    def chunk(i, _):
        row = s * NCHUNK + i
        pltpu.sync_copy(edata_hbm.at[row], srcv)
        pltpu.sync_copy(edata_hbm.at[ERWS + row], trgv)
        pltpu.sync_copy(w_hbm.at[pl.ds(row * (C // 8), C // 8)], wbuf)
        for k in range(C // L):
            srcv[pl.ds(k * L, L)] = srcv[pl.ds(k * L, L)] + off
        d0 = pltpu.async_copy(proj2_hbm.at[srcv.at[pl.ds(0, H)]],
                              projb.at[pl.ds(0, H)], semG0)
        d1 = pltpu.async_copy(proj2_hbm.at[srcv.at[pl.ds(H, H)]],
                              projb.at[pl.ds(H, H)], semG1)

        @pl.when(i > 0)
        def _():
            pltpu.make_async_copy(proj2_hbm.at[pl.ds(0, H)], numba,
                                  semS0).wait()
            pltpu.make_async_copy(proj2_hbm.at[pl.ds(0, H)], numbb,
                                  semS1).wait()
        for k in range(H // L):
            trgva[pl.ds(k * L, L)] = trgv[pl.ds(k * L, L)]
            trgvb[pl.ds(k * L, L)] = trgv[pl.ds(H + k * L, L)]

        d0.wait()
        lax.fori_loop(0, H, eb_a, 0)
        pltpu.async_copy(numba, acc_sh.at[trgva], semS0, add=True)
        d1.wait()
        lax.fori_loop(0, H, eb_b, 0)
        pltpu.async_copy(numbb, acc_sh.at[trgvb], semS1, add=True)
        return 0
    lax.fori_loop(0, NCHUNK, chunk, 0)

    pltpu.make_async_copy(proj2_hbm.at[pl.ds(0, H)], numba, semS0).wait()
    pltpu.make_async_copy(proj2_hbm.at[pl.ds(0, H)], numbb, semS1).wait()
    plsc.subcore_barrier()
    pltpu.sync_copy(acc_sh.at[pl.ds(s * PH_PT, PH_PT)],
                    num_out.at[pl.ds(c * NP2 + s * PH_PT, PH_PT)])


def _sc_n(proj2, edata, warr, hpar):
    if "n" in _SC_W_CACHE:
        return _SC_W_CACHE["n"](proj2, edata, warr, hpar)
    mesh = plsc.VectorSubcoreMesh(core_axis_name="c", subcore_axis_name="s")
    f = pl.kernel(
        _sc_n_body,
        out_type=jax.ShapeDtypeStruct((2 * NP2, 128), F32),
        mesh=mesh,
        scratch_types=[
            pltpu.VMEM((C,), jnp.int32),
            pltpu.VMEM((C,), jnp.int32),
            pltpu.VMEM((C // 2,), jnp.int32),
            pltpu.VMEM((C // 2,), jnp.int32),
            pltpu.VMEM((C // 8, 128), F32),
            pltpu.VMEM((C, 128), F32),
            pltpu.VMEM((C // 2, 128), F32),
            pltpu.VMEM((C // 2, 128), F32),
            pltpu.VMEM((16, 128), F32),
            pltpu.VMEM((128,), jnp.int32),
            pltpu.VMEM_SHARED((AROWS, 128), F32),
            pltpu.SemaphoreType.DMA,
            pltpu.SemaphoreType.DMA,
            pltpu.SemaphoreType.DMA,
            pltpu.SemaphoreType.DMA,
        ],
    )
    _SC_W_CACHE["n"] = f
    return f(proj2, edata, warr, hpar)


def _sc_gather_body(h_hbm, idx_hbm, out_hbm, idxv, rowsb, sem):
    c = lax.axis_index("c")
    s = lax.axis_index("s")
    wid = s * NCORE + c
    for j in range(2):
        base = wid * 2 + j
        pltpu.sync_copy(idx_hbm.at[base], idxv)
        pltpu.async_copy(h_hbm.at[idxv], rowsb, sem).wait()
        pltpu.sync_copy(rowsb, out_hbm.at[pl.ds(base * 128, 128)])


def _sc_gather(h, idx):
    nidx = idx.shape[0]
    idx = idx.reshape(nidx // 128, 128)
    mesh = plsc.VectorSubcoreMesh(core_axis_name="c", subcore_axis_name="s")
    f = pl.kernel(
        _sc_gather_body,
        out_type=jax.ShapeDtypeStruct((nidx, D), F32),
        mesh=mesh,
        scratch_types=[
            pltpu.VMEM((128,), jnp.int32),
            pltpu.VMEM((128, D), F32),
            pltpu.SemaphoreType.DMA,
        ],
    )
    return f(h, idx)


def _layer(h, edata, p, concat, nh):
    fout = D // nh
    W = p['W']
    ssrc_flat = p['scoring_src'].reshape(-1)
    strg_flat = p['scoring_trg'].reshape(-1)
    S = np.zeros((D, L), np.float32)
    for j in range(D):
        S[j, j // fout] = 1.0
    S = jnp.asarray(S)
    Wsrc = (W * ssrc_flat[None, :]) @ S
    Wtrg = (W * strg_flat[None, :]) @ S
    Wskip = p['W_skip'] if 'W_skip' in p else jnp.eye(h.shape[1], D, dtype=F32)

    proj2, sctab, skip, smax = _tc_pre(h, W, Wsrc, Wtrg, Wskip)

    cvec = (p['W_prob'].reshape(nh, fout) * p['scoring_prob'][0]).sum(-1)
    c16 = jnp.zeros((L,), F32).at[:nh].set(cvec)
    mh = smax[0] + smax[1] + jnp.maximum(c16, 0.0)
    mh = jnp.maximum(mh, 0.2 * mh)
    M = jnp.max(mh[:nh])
    c16 = c16 * (1.0 / 16777216.0)
    par = jnp.concatenate([c16, jnp.full((L,), M, F32),
                           jnp.zeros((128 - 2 * L,), F32)])

    warr, den2 = _sc_w(sctab, edata, par)
    f_h = 8 if nh == 16 else 0
    g_h = 1 if nh == 16 else 0
    hpar = jnp.concatenate([jnp.full((L,), f_h, jnp.int32),
                            jnp.full((L,), g_h, jnp.int32),
                            jnp.zeros((128 - 2 * L,), jnp.int32)])
    num2 = _sc_n(proj2.reshape(2 * N, 128), edata, warr, hpar)

    den2 = den2.reshape(2, NP, L)

    R = np.zeros((L, D), np.float32)
    for j in range(D):
        R[j // fout, j] = 1.0
    R = jnp.asarray(R)
    ln = 'ln_gamma' in p
    gamma = p['ln_gamma'].reshape(1, D) if ln else jnp.zeros((1, D), F32)
    beta = p['ln_beta'].reshape(1, D) if ln else jnp.zeros((1, D), F32)
    return _tc_post(num2.reshape(2, NP2, 128), den2[0], den2[1], skip, R,
                    p['bias'].reshape(1, D), gamma, beta, ln)


def kernel(node_features, edge_index_input, edge_prob_input, x, params):
    npad = EPAD - E
    src = jnp.concatenate([edge_index_input[0].astype(jnp.int32),
                           jnp.zeros((npad,), jnp.int32)])
    trg = jnp.concatenate([edge_index_input[1].astype(jnp.int32),
                           jnp.full((npad,), NP - 1, jnp.int32)])
    prob = jnp.concatenate([edge_prob_input[:, 0],
                            jnp.zeros((npad,), F32)])
    pfix = (prob * 16777216.0).astype(jnp.int32)
    edata = jnp.concatenate([
        src.reshape(ERWS, C), trg.reshape(ERWS, C), pfix.reshape(ERWS, C),
        jnp.zeros((EDROWS - 3 * ERWS, C), jnp.int32)])

    h = _layer(node_features, edata, params['enc'], True, 16)
    h = _layer(h, edata, params['gm0'], True, 16)
    h = _layer(h, edata, params['gm1'], False, 1)

    B, S_ = x.shape
    idx = x.reshape(-1).astype(jnp.int32)
    out = _sc_gather(h, idx)
    return out.reshape(B, S_, D)
